# baseline (speedup 1.0000x reference)
import functools

import jax
import jax.numpy as jnp
from jax import lax
from jax.experimental import pallas as pl
from jax.experimental.pallas import tpu as pltpu
from jax.experimental.pallas import tpu_sc as plsc

NC = 2
NS = 16
NT = NC * NS
LANES = 16

N = 10000
D = 128
E = 320000
EPT = E // NT
NSL = EPT // LANES
GRP = 128
NG = (EPT + GRP - 1) // GRP
DUMP = N * N
TSZ = N * N + 16
PEPT = NG * GRP
CSL = 656
TOTSLOT = 100
HALF = 5000
HROWS = 5120
G2 = 64
NG2 = PEPT // G2
ASL = 160

_mesh = plsc.VectorSubcoreMesh(core_axis_name="c", subcore_axis_name="s")


def _wid():
    return lax.axis_index("c") * NS + lax.axis_index("s")


def _iota16():
    return lax.broadcasted_iota(jnp.int32, (LANES,), 0)


def _compute_keys(s_v, d_v, k_v):
    def body(i, _):
        g = i // 8
        o = (i % 8) * LANES
        s16 = s_v[pl.ds(i * LANES, LANES)]
        d16 = d_v[pl.ds(i * LANES, LANES)]
        k_v[g, pl.ds(o, LANES)] = s16 * N + d16
        return 0
    lax.fori_loop(0, NSL, body, 0)
    dump16 = jnp.full((LANES,), DUMP, jnp.int32)
    for l in range(1, 8):
        k_v[NG - 1, pl.ds(l * LANES, LANES)] = dump16


def _load_pair(pair_hbm, base, s_v, d_v):
    pltpu.sync_copy(pair_hbm.at[pl.ds(base, EPT)], s_v)
    pltpu.sync_copy(pair_hbm.at[pl.ds(E + base, EPT)], d_v)


def _gather_rows(t_hbm, k_v, o_v, sem):
    def fire(g, _):
        pltpu.async_copy(t_hbm.at[k_v.at[g]], o_v.at[g], sem)
        return 0
    lax.fori_loop(0, NG, fire, 0)
    def drain(g, _):
        pltpu.make_async_copy(t_hbm.at[pl.ds(0, GRP)], o_v.at[g], sem).wait()
        return 0
    lax.fori_loop(0, NG, drain, 0)


def _mask_kernel(t_hbm, mask_hbm, tok_out, s_v, d_v, k_v, val_v, sem):
    base = _wid() * EPT
    _load_pair(mask_hbm, base, s_v, d_v)
    _compute_keys(s_v, d_v, k_v)
    m2 = jnp.full((LANES,), -2, jnp.int32)
    for l in range(8):
        val_v[pl.ds(l * LANES, LANES)] = m2
    def fire(g, _):
        pltpu.async_copy(val_v, t_hbm.at[k_v.at[g]], sem)
        return 0
    lax.fori_loop(0, NG, fire, 0)
    def drain(g, _):
        pltpu.make_async_copy(t_hbm.at[pl.ds(0, GRP)], k_v.at[g], sem).wait()
        return 0
    lax.fori_loop(0, NG, drain, 0)
    @pl.when(_wid() == 0)
    def _():
        val_v[pl.ds(0, LANES)] = jnp.zeros((LANES,), jnp.int32)
        pltpu.sync_copy(val_v.at[pl.ds(0, LANES)], tok_out)


def _build_e1():
    return pl.kernel(
        _mask_kernel,
        out_type=jax.ShapeDtypeStruct((LANES,), jnp.int32),
        mesh=_mesh,
        scratch_types=[
            pltpu.VMEM((EPT,), jnp.int32),
            pltpu.VMEM((EPT,), jnp.int32),
            pltpu.VMEM((NG, GRP), jnp.int32),
            pltpu.VMEM((GRP,), jnp.int32),
            pltpu.SemaphoreType.DMA,
        ],
    )


def _claim_kernel(id_base, t_hbm, ei_hbm, tok_in, tok_out,
                  s_v, d_v, k_v, m_v, q_v, id_v, sem):
    del tok_in
    wid = _wid()
    base = wid * EPT
    _load_pair(ei_hbm, base, s_v, d_v)
    _compute_keys(s_v, d_v, k_v)
    _gather_rows(t_hbm, k_v, m_v, sem)
    iota = _iota16()
    def sel(i, _):
        g = i // 8
        o = (i % 8) * LANES
        m16 = m_v[g, pl.ds(o, LANES)]
        k16 = k_v[g, pl.ds(o, LANES)]
        q_v[g, pl.ds(o, LANES)] = jnp.where(m16 != -1, k16, DUMP)
        p = i * LANES + iota
        id_v[g, pl.ds(o, LANES)] = id_base + base + p
        return 0
    lax.fori_loop(0, NG * 8, sel, 0)
    def fire2(g, _):
        pltpu.async_copy(id_v.at[g], t_hbm.at[q_v.at[g]], sem)
        return 0
    lax.fori_loop(0, NG, fire2, 0)
    def drain2(g, _):
        pltpu.make_async_copy(t_hbm.at[pl.ds(0, GRP)], m_v.at[g], sem).wait()
        return 0
    lax.fori_loop(0, NG, drain2, 0)
    @pl.when(wid == 0)
    def _():
        id_v[0, pl.ds(0, LANES)] = jnp.zeros((LANES,), jnp.int32)
        pltpu.sync_copy(id_v.at[0, pl.ds(0, LANES)], tok_out)


def _build_claim(id_base):
    return pl.kernel(
        functools.partial(_claim_kernel, id_base),
        out_type=jax.ShapeDtypeStruct((LANES,), jnp.int32),
        mesh=_mesh,
        scratch_types=[
            pltpu.VMEM((EPT,), jnp.int32),
            pltpu.VMEM((EPT,), jnp.int32),
            pltpu.VMEM((NG, GRP), jnp.int32),
            pltpu.VMEM((NG, GRP), jnp.int32),
            pltpu.VMEM((NG, GRP), jnp.int32),
            pltpu.VMEM((NG, GRP), jnp.int32),
            pltpu.SemaphoreType.DMA,
        ],
    )


def _compact_kernel(id_base, t_hbm, ei_hbm, tok_in, src_out, dst_out, val_out,
                    cnt_out, tcnt_out, s_v, d_v, k_v, o_v, f_v, sidx_v,
                    cs_v, cd_v, cf_v, zc_v, gcnt_v, cnt_v, cnts_sp, sem):
    del tok_in
    wid = _wid()
    tile = lax.axis_index("s")
    base = wid * EPT
    _load_pair(ei_hbm, base, s_v, d_v)
    _compute_keys(s_v, d_v, k_v)
    _gather_rows(t_hbm, k_v, o_v, sem)
    z16 = jnp.zeros((LANES,), jnp.int32)
    def zc(i, _):
        zc_v[pl.ds(i * LANES, LANES)] = z16
        return 0
    lax.fori_loop(0, CSL // LANES, zc, 0)
    cbase = tile * CSL
    pltpu.sync_copy(zc_v, cnts_sp.at[pl.ds(cbase, CSL)])
    iota = _iota16()
    def flg(i, _):
        g = i // 8
        o = (i % 8) * LANES
        o16 = o_v[g, pl.ds(o, LANES)]
        won = o16 == (id_base + base + i * LANES + iota)
        f_v[g, pl.ds(o, LANES)] = jnp.where(won, 1, 0)
        sidx_v[g, pl.ds(o, LANES)] = z16 + (cbase + g)
        return 0
    lax.fori_loop(0, NSL, flg, 0)
    for l in range(1, 8):
        f_v[NG - 1, pl.ds(l * LANES, LANES)] = z16
        sidx_v[NG - 1, pl.ds(l * LANES, LANES)] = z16 + (cbase + NG - 1)
    def tot_idx(i, _):
        o_v[i // 8, pl.ds((i % 8) * LANES, LANES)] = z16 + (cbase + TOTSLOT)
        return 0
    lax.fori_loop(0, NG * 8, tot_idx, 0)
    def cfire(g, _):
        pltpu.async_copy(f_v.at[g], cnts_sp.at[sidx_v.at[g]], sem, add=True)
        pltpu.async_copy(f_v.at[g], cnts_sp.at[o_v.at[g]], sem, add=True)
        return 0
    lax.fori_loop(0, NG, cfire, 0)
    def cdrain(g, _):
        pltpu.make_async_copy(t_hbm.at[pl.ds(0, GRP)], k_v.at[g], sem).wait()
        pltpu.make_async_copy(t_hbm.at[pl.ds(0, GRP)], k_v.at[g], sem).wait()
        return 0
    lax.fori_loop(0, NG, cdrain, 0)
    pltpu.sync_copy(cnts_sp.at[pl.ds(cbase, GRP)], gcnt_v)
    gslices = [gcnt_v[pl.ds(k * LANES, LANES)] for k in range(8)]
    pos = jnp.int32(0)
    for g in range(NG):
        gc = gslices[g // LANES][g % LANES]
        @pl.when(gc > 0)
        def _(g=g, pos=pos):
            for l in range(8):
                o = l * LANES
                cs_v[pl.ds(pos * GRP + o, LANES)] = (
                    s_v[pl.ds(g * GRP + o, LANES)]
                    if g * GRP + o + LANES <= EPT
                    else z16)
                cd_v[pl.ds(pos * GRP + o, LANES)] = (
                    d_v[pl.ds(g * GRP + o, LANES)]
                    if g * GRP + o + LANES <= EPT
                    else z16)
                cf_v[pl.ds(pos * GRP + o, LANES)] = f_v[g, pl.ds(o, LANES)]
        pos = pos + jnp.where(gc > 0, 1, 0)
    pbase = wid * PEPT
    pltpu.sync_copy(cs_v.at[pl.ds(0, PEPT)], src_out.at[pl.ds(pbase, PEPT)])
    pltpu.sync_copy(cd_v.at[pl.ds(0, PEPT)], dst_out.at[pl.ds(pbase, PEPT)])
    pltpu.sync_copy(cf_v.at[pl.ds(0, PEPT)], val_out.at[pl.ds(pbase, PEPT)])
    cnt_v[pl.ds(0, LANES)] = z16 + pos * GRP
    pltpu.sync_copy(cnt_v, cnt_out.at[wid])
    tslice = gcnt_v[pl.ds((TOTSLOT // LANES) * LANES, LANES)]
    cnt_v[pl.ds(0, LANES)] = z16 + tslice[TOTSLOT % LANES]
    pltpu.sync_copy(cnt_v, tcnt_out.at[wid])


def _build_compact(id_base):
    return pl.kernel(
        functools.partial(_compact_kernel, id_base),
        out_type=(
            jax.ShapeDtypeStruct((NT * PEPT,), jnp.int32),
            jax.ShapeDtypeStruct((NT * PEPT,), jnp.int32),
            jax.ShapeDtypeStruct((NT * PEPT,), jnp.int32),
            jax.ShapeDtypeStruct((NT, LANES), jnp.int32),
            jax.ShapeDtypeStruct((NT, LANES), jnp.int32),
        ),
        mesh=_mesh,
        scratch_types=[
            pltpu.VMEM((EPT,), jnp.int32),
            pltpu.VMEM((EPT,), jnp.int32),
            pltpu.VMEM((NG, GRP), jnp.int32),
            pltpu.VMEM((NG, GRP), jnp.int32),
            pltpu.VMEM((NG, GRP), jnp.int32),
            pltpu.VMEM((NG, GRP), jnp.int32),
            pltpu.VMEM((PEPT,), jnp.int32),
            pltpu.VMEM((PEPT,), jnp.int32),
            pltpu.VMEM((PEPT,), jnp.int32),
            pltpu.VMEM((CSL,), jnp.int32),
            pltpu.VMEM((GRP,), jnp.int32),
            pltpu.VMEM((LANES,), jnp.int32),
            pltpu.VMEM_SHARED((NS * CSL,), jnp.int32),
            pltpu.SemaphoreType.DMA,
        ],
    )


def _zero_zb(zb_v):
    z16 = jnp.zeros((LANES,), jnp.float32)
    def body(i, _):
        zb_v[i // 8, pl.ds((i % 8) * LANES, LANES)] = z16
        return 0
    lax.fori_loop(0, 8 * 8, body, 0)


def _zero_acc(acc, tile, zb_v):
    start = tile * (HROWS // NS)
    for j in range(HROWS // NS // 8):
        pltpu.sync_copy(zb_v, acc.at[pl.ds(start + j * 8, 8)])


def _get_cnt(cnt_hbm, r, cnt_v):
    pltpu.sync_copy(cnt_hbm.at[r], cnt_v)
    return cnt_v[pl.ds(0, LANES)][0]


def _prop_kernel(x_hbm, src_hbm, dst_hbm, val_hbm, cnt_hbm, out_hbm,
                 s_g, d_g, f_g, ss_v, sds_v, rows_v, zb_v, cnt_v, acc, sem):
    core = lax.axis_index("c")
    tile = lax.axis_index("s")
    _zero_zb(zb_v)
    r = core * NS + tile
    cnt = _get_cnt(cnt_hbm, r, cnt_v)
    pbase = r * PEPT
    dumprow = HALF + tile * 4
    for half in range(2):
        lo = half * HALF
        _zero_acc(acc, tile, zb_v)
        plsc.subcore_barrier()
        def grp(g, _):
            @pl.when(g * GRP < cnt)
            def _():
                pltpu.sync_copy(src_hbm.at[pl.ds(pbase + g * GRP, GRP)], s_g)
                pltpu.sync_copy(dst_hbm.at[pl.ds(pbase + g * GRP, GRP)], d_g)
                pltpu.sync_copy(val_hbm.at[pl.ds(pbase + g * GRP, GRP)], f_g)
                for l in range(8):
                    sl = pl.ds(l * LANES, LANES)
                    fl = f_g[sl] > 0
                    s16 = s_g[sl]
                    d16 = d_g[sl]
                    ok = fl & (d16 >= lo) & (d16 < lo + HALF)
                    ss_v[sl] = jnp.where(fl, s16, 0)
                    sds_v[sl] = jnp.where(ok, d16 - lo, dumprow)
                pltpu.async_copy(x_hbm.at[ss_v], rows_v, sem).wait()
                pltpu.sync_copy(rows_v, acc.at[sds_v], add=True)
            return 0
        lax.fori_loop(0, NG, grp, 0)
        plsc.subcore_barrier()
        rt = HROWS // NS
        pltpu.sync_copy(
            acc.at[pl.ds(tile * rt, rt)],
            out_hbm.at[core, pl.ds(half * HROWS + tile * rt, rt)])
        plsc.subcore_barrier()


def _build_prop():
    return pl.kernel(
        _prop_kernel,
        out_type=jax.ShapeDtypeStruct((NC, 2 * HROWS, D), jnp.float32),
        mesh=_mesh,
        scratch_types=[
            pltpu.VMEM((GRP,), jnp.int32),
            pltpu.VMEM((GRP,), jnp.int32),
            pltpu.VMEM((GRP,), jnp.int32),
            pltpu.VMEM((GRP,), jnp.int32),
            pltpu.VMEM((GRP,), jnp.int32),
            pltpu.VMEM((GRP, D), jnp.float32),
            pltpu.VMEM((8, D), jnp.float32),
            pltpu.VMEM((LANES,), jnp.int32),
            pltpu.VMEM_SHARED((HROWS, D), jnp.float32),
            pltpu.SemaphoreType.DMA,
        ],
    )


def _attn_kernel(x_hbm, xq_hbm, x2_hbm, src_hbm, dst_hbm, val_hbm, cnt_hbm,
                 out_hbm, s_g, d_g, f_g, ss_v, sdg_v, sds_v,
                 xqr_v, xdr_v, x2r_v, ra_v, att16_v, idxr_v, za_v, zb_v,
                 cnt_v, acc, attn_sp, sem):
    core = lax.axis_index("c")
    tile = lax.axis_index("s")
    _zero_zb(zb_v)
    z16f = jnp.zeros((LANES,), jnp.float32)
    def za(i, _):
        za_v[pl.ds(i * LANES, LANES)] = z16f
        return 0
    lax.fori_loop(0, ASL // LANES, za, 0)
    abase = tile * ASL
    z16i = jnp.zeros((LANES,), jnp.int32)
    def ip(i, _):
        b = i // 8
        o = i % 8
        idxr_v[b, pl.ds(o * LANES, LANES)] = z16i + (abase + 8 * b + o)
        return 0
    lax.fori_loop(0, 8 * 8, ip, 0)
    r = core * NS + tile
    cnt = _get_cnt(cnt_hbm, r, cnt_v)
    pbase = r * PEPT
    dumprow = HALF + tile * 4
    for half in range(2):
        lo = half * HALF
        _zero_acc(acc, tile, zb_v)
        plsc.subcore_barrier()
        def grp(g, _):
            @pl.when(g * G2 < cnt)
            def _():
                pltpu.sync_copy(src_hbm.at[pl.ds(pbase + g * G2, G2)], s_g)
                pltpu.sync_copy(dst_hbm.at[pl.ds(pbase + g * G2, G2)], d_g)
                pltpu.sync_copy(val_hbm.at[pl.ds(pbase + g * G2, G2)], f_g)
                for l in range(4):
                    sl = pl.ds(l * LANES, LANES)
                    fl = f_g[sl] > 0
                    s16 = s_g[sl]
                    d16 = d_g[sl]
                    ok = fl & (d16 >= lo) & (d16 < lo + HALF)
                    ss_v[sl] = jnp.where(fl, s16, 0)
                    sdg_v[sl] = jnp.where(fl, d16, 0)
                    sds_v[sl] = jnp.where(ok, d16 - lo, dumprow)
                cp1 = pltpu.async_copy(xq_hbm.at[ss_v], xqr_v, sem)
                cp2 = pltpu.async_copy(x_hbm.at[sdg_v], xdr_v, sem)
                cp3 = pltpu.async_copy(x2_hbm.at[ss_v], x2r_v, sem)
                cp1.wait()
                cp2.wait()
                cp3.wait()
                def prod(rr, _):
                    a16 = (xqr_v[rr, pl.ds(0, LANES)]
                           * xdr_v[rr, pl.ds(0, LANES)])
                    for l in range(1, 8):
                        a16 = a16 + (xqr_v[rr, pl.ds(l * LANES, LANES)]
                                     * xdr_v[rr, pl.ds(l * LANES, LANES)])
                    ra_v[pl.ds(rr * LANES, LANES)] = a16
                    return 0
                lax.fori_loop(0, G2, prod, 0)
                pltpu.sync_copy(za_v, attn_sp.at[pl.ds(abase, ASL)])
                def afire(b, _):
                    pltpu.async_copy(ra_v.at[pl.ds(b * GRP, GRP)],
                                     attn_sp.at[idxr_v.at[b]], sem, add=True)
                    return 0
                lax.fori_loop(0, 8, afire, 0)
                def adrain(b, _):
                    pltpu.make_async_copy(x_hbm.at[0], xdr_v.at[0], sem).wait()
                    return 0
                lax.fori_loop(0, 8, adrain, 0)
                def bfire(b, _):
                    pltpu.async_copy(attn_sp.at[idxr_v.at[b]],
                                     att16_v.at[pl.ds(b * GRP, GRP)], sem)
                    return 0
                lax.fori_loop(0, 8, bfire, 0)
                def bdrain(b, _):
                    pltpu.make_async_copy(x_hbm.at[0], xdr_v.at[0], sem).wait()
                    return 0
                lax.fori_loop(0, 8, bdrain, 0)
                def ymul(rr, _):
                    at16 = att16_v[pl.ds(rr * LANES, LANES)]
                    for l in range(8):
                        xdr_v[rr, pl.ds(l * LANES, LANES)] = (
                            at16 * x2r_v[rr, pl.ds(l * LANES, LANES)])
                    return 0
                lax.fori_loop(0, G2, ymul, 0)
                pltpu.sync_copy(xdr_v, acc.at[sds_v], add=True)
            return 0
        lax.fori_loop(0, NG2, grp, 0)
        plsc.subcore_barrier()
        rt = HROWS // NS
        pltpu.sync_copy(
            acc.at[pl.ds(tile * rt, rt)],
            out_hbm.at[core, pl.ds(half * HROWS + tile * rt, rt)])
        plsc.subcore_barrier()


def _build_attn():
    return pl.kernel(
        _attn_kernel,
        out_type=jax.ShapeDtypeStruct((NC, 2 * HROWS, D), jnp.float32),
        mesh=_mesh,
        scratch_types=[
            pltpu.VMEM((G2,), jnp.int32),
            pltpu.VMEM((G2,), jnp.int32),
            pltpu.VMEM((G2,), jnp.int32),
            pltpu.VMEM((G2,), jnp.int32),
            pltpu.VMEM((G2,), jnp.int32),
            pltpu.VMEM((G2,), jnp.int32),
            pltpu.VMEM((G2, D), jnp.float32),
            pltpu.VMEM((G2, D), jnp.float32),
            pltpu.VMEM((G2, D), jnp.float32),
            pltpu.VMEM((G2 * LANES,), jnp.float32),
            pltpu.VMEM((G2 * LANES,), jnp.float32),
            pltpu.VMEM((8, GRP), jnp.int32),
            pltpu.VMEM((ASL,), jnp.float32),
            pltpu.VMEM((8, D), jnp.float32),
            pltpu.VMEM((LANES,), jnp.int32),
            pltpu.VMEM_SHARED((HROWS, D), jnp.float32),
            pltpu.VMEM_SHARED((NS * ASL,), jnp.float32),
            pltpu.SemaphoreType.DMA,
        ],
    )


_RB = 1000


def _t1_body(x_ref, p0_ref, p1_ref, w1_ref, q_ref, w2_ref,
             x1_ref, xq_ref, x2_ref):
    x = x_ref[...]
    p = p0_ref[...] + p1_ref[...]
    w1 = w1_ref[...]
    q = q_ref[...]
    h = jnp.dot(p, w1.T, preferred_element_type=jnp.float32)
    x1_ref[...] = x + jnp.dot(h, q, preferred_element_type=jnp.float32) * (
        float(N) / float(E))
    xq_ref[...] = jnp.dot(x, q, preferred_element_type=jnp.float32)
    x2_ref[...] = jnp.dot(x, w2_ref[...].T, preferred_element_type=jnp.float32)


def _t1(x, p0, p1, w1, q, w2):
    full = pl.BlockSpec((D, D), lambda i: (0, 0))
    blk = pl.BlockSpec((_RB, D), lambda i: (i, 0))
    return pl.pallas_call(
        _t1_body,
        grid=(N // _RB,),
        in_specs=[blk, blk, blk, full, full, full],
        out_specs=[blk, blk, blk],
        out_shape=[jax.ShapeDtypeStruct((N, D), jnp.float32)] * 3,
    )(x, p0, p1, w1, q, w2)


def _t2_body(x1_ref, a0_ref, a1_ref, q_ref, cnt_ref, out_ref):
    a = a0_ref[...] + a1_ref[...]
    cnt2 = jnp.sum(cnt_ref[...][:, 0]).astype(jnp.float32)
    scale = float(N) / cnt2
    out_ref[...] = x1_ref[...] - jnp.dot(
        a, q_ref[...], preferred_element_type=jnp.float32) * scale


def _t2(x1, a0, a1, q, cnt2):
    full = pl.BlockSpec((D, D), lambda i: (0, 0))
    blk = pl.BlockSpec((_RB, D), lambda i: (i, 0))
    cblk = pl.BlockSpec((NT, LANES), lambda i: (0, 0))
    return pl.pallas_call(
        _t2_body,
        grid=(N // _RB,),
        in_specs=[blk, blk, blk, full, cblk],
        out_specs=blk,
        out_shape=jax.ShapeDtypeStruct((N, D), jnp.float32),
    )(x1, a0, a1, q, cnt2)


def _halves(arr):
    return (arr[:, :HALF], arr[:, HROWS:HROWS + HALF])


def kernel(input, edge_index, edge_index_2, Q, mask, W1_0, W2_0, W1_1, W2_1):
    t = jnp.full((TSZ,), -1, jnp.int32)
    mask_f = mask.reshape(-1)
    e1_f = edge_index.reshape(-1)
    e2_f = edge_index_2.reshape(-1)
    tok1 = _build_e1()(t, mask_f)
    tok2 = _build_claim(0)(t, e1_f, tok1)
    src1, dst1, val1, cnt1, _tc1 = _build_compact(0)(t, e1_f, tok2)
    tok4 = _build_claim(E)(t, e2_f, cnt1)
    src2, dst2, val2, cnt2, tcnt2 = _build_compact(E)(t, e2_f, tok4)

    prop_k = _build_prop()
    attn_k = _build_attn()

    x = input
    for (w1, w2) in ((W1_0, W2_0), (W1_1, W2_1)):
        prop = prop_k(x, src1, dst1, val1, cnt1)
        pa, pb = _halves(prop)
        p0 = jnp.concatenate([pa[0], pb[0]], axis=0)
        p1 = jnp.concatenate([pa[1], pb[1]], axis=0)
        x1, xq, x2 = _t1(x, p0, p1, w1, Q, w2)
        agg = attn_k(x, xq, x2, src2, dst2, val2, cnt2)
        aa, ab = _halves(agg)
        a0 = jnp.concatenate([aa[0], ab[0]], axis=0)
        a1 = jnp.concatenate([aa[1], ab[1]], axis=0)
        x = _t2(x1, a0, a1, Q, tcnt2)
    return x


# speedup vs baseline: 15.1761x; 15.1761x over previous
import functools

import jax
import jax.numpy as jnp
from jax import lax
from jax.experimental import pallas as pl
from jax.experimental.pallas import tpu as pltpu
from jax.experimental.pallas import tpu_sc as plsc

NC = 2
NS = 16
NT = NC * NS
LANES = 16

N = 10000
D = 128
E = 320000
EPT = E // NT
NSL = EPT // LANES
GRP = 128
NG = (EPT + GRP - 1) // GRP
DUMP = N * N
TSZ = N * N + NT * GRP + 64
PEPT = NG * GRP
CSL = 656
TOTSLOT = 100
HALF = 5000
HROWS = 5120
G2 = 64
NG2 = PEPT // G2
ASL = 160

_mesh = plsc.VectorSubcoreMesh(core_axis_name="c", subcore_axis_name="s")


def _wid():
    return lax.axis_index("c") * NS + lax.axis_index("s")


def _iota16():
    return lax.broadcasted_iota(jnp.int32, (LANES,), 0)


def _dump16(wid, l):
    return DUMP + wid * GRP + l * LANES + _iota16()


def _compute_keys(wid, s_v, d_v, k_v):
    def body(i, _):
        g = i // 8
        o = (i % 8) * LANES
        s16 = s_v[pl.ds(i * LANES, LANES)]
        d16 = d_v[pl.ds(i * LANES, LANES)]
        k_v[g, pl.ds(o, LANES)] = s16 * N + d16
        return 0
    lax.fori_loop(0, NSL, body, 0)
    for l in range(1, 8):
        k_v[NG - 1, pl.ds(l * LANES, LANES)] = _dump16(wid, l)


def _load_pair(pair_hbm, base, s_v, d_v):
    pltpu.sync_copy(pair_hbm.at[pl.ds(base, EPT)], s_v)
    pltpu.sync_copy(pair_hbm.at[pl.ds(E + base, EPT)], d_v)


def _gather_rows(t_hbm, k_v, o_v, sem):
    def fire(g, _):
        pltpu.async_copy(t_hbm.at[k_v.at[g]], o_v.at[g], sem)
        return 0
    lax.fori_loop(0, NG, fire, 0)
    def drain(g, _):
        pltpu.make_async_copy(t_hbm.at[pl.ds(0, GRP)], o_v.at[g], sem).wait()
        return 0
    lax.fori_loop(0, NG, drain, 0)


def _mask_kernel(t_hbm, mask_hbm, tok_out, s_v, d_v, k_v, val_v, sem):
    wid = _wid()
    base = wid * EPT
    _load_pair(mask_hbm, base, s_v, d_v)
    _compute_keys(wid, s_v, d_v, k_v)
    m2 = jnp.full((LANES,), -2, jnp.int32)
    for l in range(8):
        val_v[pl.ds(l * LANES, LANES)] = m2
    def fire(g, _):
        pltpu.async_copy(val_v, t_hbm.at[k_v.at[g]], sem)
        return 0
    lax.fori_loop(0, NG, fire, 0)
    def drain(g, _):
        pltpu.make_async_copy(t_hbm.at[pl.ds(0, GRP)], k_v.at[g], sem).wait()
        return 0
    lax.fori_loop(0, NG, drain, 0)
    @pl.when(_wid() == 0)
    def _():
        val_v[pl.ds(0, LANES)] = jnp.zeros((LANES,), jnp.int32)
        pltpu.sync_copy(val_v.at[pl.ds(0, LANES)], tok_out)


def _build_e1():
    return pl.kernel(
        _mask_kernel,
        out_type=jax.ShapeDtypeStruct((LANES,), jnp.int32),
        mesh=_mesh,
        scratch_types=[
            pltpu.VMEM((EPT,), jnp.int32),
            pltpu.VMEM((EPT,), jnp.int32),
            pltpu.VMEM((NG, GRP), jnp.int32),
            pltpu.VMEM((GRP,), jnp.int32),
            pltpu.SemaphoreType.DMA,
        ],
    )


def _claim_kernel(id_base, t_hbm, ei_hbm, tok_in, tok_out,
                  s_v, d_v, k_v, m_v, q_v, id_v, sem):
    del tok_in
    wid = _wid()
    base = wid * EPT
    _load_pair(ei_hbm, base, s_v, d_v)
    _compute_keys(wid, s_v, d_v, k_v)
    _gather_rows(t_hbm, k_v, m_v, sem)
    iota = _iota16()
    def sel(i, _):
        g = i // 8
        o = (i % 8) * LANES
        m16 = m_v[g, pl.ds(o, LANES)]
        k16 = k_v[g, pl.ds(o, LANES)]
        q_v[g, pl.ds(o, LANES)] = jnp.where(
            m16 != -1, k16, DUMP + wid * GRP + o + iota)
        p = i * LANES + iota
        id_v[g, pl.ds(o, LANES)] = id_base + base + p
        return 0
    lax.fori_loop(0, NSL, sel, 0)
    m4 = jnp.full((LANES,), -4, jnp.int32)
    for l in range(1, 8):
        q_v[NG - 1, pl.ds(l * LANES, LANES)] = _dump16(wid, l)
        id_v[NG - 1, pl.ds(l * LANES, LANES)] = m4
    def fire2(g, _):
        pltpu.async_copy(id_v.at[g], t_hbm.at[q_v.at[g]], sem)
        return 0
    lax.fori_loop(0, NG, fire2, 0)
    def drain2(g, _):
        pltpu.make_async_copy(t_hbm.at[pl.ds(0, GRP)], m_v.at[g], sem).wait()
        return 0
    lax.fori_loop(0, NG, drain2, 0)
    @pl.when(wid == 0)
    def _():
        id_v[0, pl.ds(0, LANES)] = jnp.zeros((LANES,), jnp.int32)
        pltpu.sync_copy(id_v.at[0, pl.ds(0, LANES)], tok_out)


def _build_claim(id_base):
    return pl.kernel(
        functools.partial(_claim_kernel, id_base),
        out_type=jax.ShapeDtypeStruct((LANES,), jnp.int32),
        mesh=_mesh,
        scratch_types=[
            pltpu.VMEM((EPT,), jnp.int32),
            pltpu.VMEM((EPT,), jnp.int32),
            pltpu.VMEM((NG, GRP), jnp.int32),
            pltpu.VMEM((NG, GRP), jnp.int32),
            pltpu.VMEM((NG, GRP), jnp.int32),
            pltpu.VMEM((NG, GRP), jnp.int32),
            pltpu.SemaphoreType.DMA,
        ],
    )


def _compact_kernel(id_base, t_hbm, ei_hbm, tok_in, src_out, dst_out, val_out,
                    cnt_out, tcnt_out, s_v, d_v, k_v, o_v, f_v, sidx_v,
                    cs_v, cd_v, cf_v, zc_v, gcnt_v, cnt_v, cnts_sp, sem):
    del tok_in
    wid = _wid()
    tile = lax.axis_index("s")
    base = wid * EPT
    _load_pair(ei_hbm, base, s_v, d_v)
    _compute_keys(wid, s_v, d_v, k_v)
    _gather_rows(t_hbm, k_v, o_v, sem)
    z16 = jnp.zeros((LANES,), jnp.int32)
    def zc(i, _):
        zc_v[pl.ds(i * LANES, LANES)] = z16
        return 0
    lax.fori_loop(0, CSL // LANES, zc, 0)
    cbase = tile * CSL
    pltpu.sync_copy(zc_v, cnts_sp.at[pl.ds(cbase, CSL)])
    iota = _iota16()
    def flg(i, _):
        g = i // 8
        o = (i % 8) * LANES
        o16 = o_v[g, pl.ds(o, LANES)]
        won = o16 == (id_base + base + i * LANES + iota)
        f_v[g, pl.ds(o, LANES)] = jnp.where(won, 1, 0)
        sidx_v[g, pl.ds(o, LANES)] = z16 + (cbase + g)
        return 0
    lax.fori_loop(0, NSL, flg, 0)
    for l in range(1, 8):
        f_v[NG - 1, pl.ds(l * LANES, LANES)] = z16
        sidx_v[NG - 1, pl.ds(l * LANES, LANES)] = z16 + (cbase + NG - 1)
    def cfire(g, _):
        pltpu.async_copy(f_v.at[g], cnts_sp.at[sidx_v.at[g]], sem, add=True)
        return 0
    lax.fori_loop(0, NG, cfire, 0)
    def cdrain(g, _):
        pltpu.make_async_copy(t_hbm.at[pl.ds(0, GRP)], k_v.at[g], sem).wait()
        return 0
    lax.fori_loop(0, NG, cdrain, 0)
    pltpu.sync_copy(cnts_sp.at[pl.ds(cbase, GRP)], gcnt_v)
    iota = _iota16()
    for l in range(8):
        o = l * LANES
        gv = gcnt_v[pl.ds(o, LANES)]
        o_v[0, pl.ds(o, LANES)] = jnp.where(o + iota < NG, gv, 0)
        sidx_v[0, pl.ds(o, LANES)] = z16 + (cbase + TOTSLOT)
    pltpu.sync_copy(o_v.at[0], cnts_sp.at[sidx_v.at[0]], add=True)
    pltpu.sync_copy(cnts_sp.at[pl.ds(cbase, GRP)], gcnt_v)
    gslices = [gcnt_v[pl.ds(k * LANES, LANES)] for k in range(8)]
    pos = jnp.int32(0)
    for g in range(NG):
        gc = gslices[g // LANES][g % LANES]
        @pl.when(gc > 0)
        def _(g=g, pos=pos):
            for l in range(8):
                o = l * LANES
                cs_v[pl.ds(pos * GRP + o, LANES)] = (
                    s_v[pl.ds(g * GRP + o, LANES)]
                    if g * GRP + o + LANES <= EPT
                    else z16)
                cd_v[pl.ds(pos * GRP + o, LANES)] = (
                    d_v[pl.ds(g * GRP + o, LANES)]
                    if g * GRP + o + LANES <= EPT
                    else z16)
                cf_v[pl.ds(pos * GRP + o, LANES)] = f_v[g, pl.ds(o, LANES)]
        pos = pos + jnp.where(gc > 0, 1, 0)
    pbase = wid * PEPT
    pltpu.sync_copy(cs_v.at[pl.ds(0, PEPT)], src_out.at[pl.ds(pbase, PEPT)])
    pltpu.sync_copy(cd_v.at[pl.ds(0, PEPT)], dst_out.at[pl.ds(pbase, PEPT)])
    pltpu.sync_copy(cf_v.at[pl.ds(0, PEPT)], val_out.at[pl.ds(pbase, PEPT)])
    cnt_v[pl.ds(0, LANES)] = z16 + pos * GRP
    pltpu.sync_copy(cnt_v, cnt_out.at[wid])
    tslice = gcnt_v[pl.ds((TOTSLOT // LANES) * LANES, LANES)]
    cnt_v[pl.ds(0, LANES)] = z16 + tslice[TOTSLOT % LANES]
    pltpu.sync_copy(cnt_v, tcnt_out.at[wid])


def _build_compact(id_base):
    return pl.kernel(
        functools.partial(_compact_kernel, id_base),
        out_type=(
            jax.ShapeDtypeStruct((NT * PEPT,), jnp.int32),
            jax.ShapeDtypeStruct((NT * PEPT,), jnp.int32),
            jax.ShapeDtypeStruct((NT * PEPT,), jnp.int32),
            jax.ShapeDtypeStruct((NT, LANES), jnp.int32),
            jax.ShapeDtypeStruct((NT, LANES), jnp.int32),
        ),
        mesh=_mesh,
        scratch_types=[
            pltpu.VMEM((EPT,), jnp.int32),
            pltpu.VMEM((EPT,), jnp.int32),
            pltpu.VMEM((NG, GRP), jnp.int32),
            pltpu.VMEM((NG, GRP), jnp.int32),
            pltpu.VMEM((NG, GRP), jnp.int32),
            pltpu.VMEM((NG, GRP), jnp.int32),
            pltpu.VMEM((PEPT,), jnp.int32),
            pltpu.VMEM((PEPT,), jnp.int32),
            pltpu.VMEM((PEPT,), jnp.int32),
            pltpu.VMEM((CSL,), jnp.int32),
            pltpu.VMEM((GRP,), jnp.int32),
            pltpu.VMEM((LANES,), jnp.int32),
            pltpu.VMEM_SHARED((NS * CSL,), jnp.int32),
            pltpu.SemaphoreType.DMA,
        ],
    )


def _zero_zb(zb_v):
    z16 = jnp.zeros((LANES,), jnp.float32)
    def body(i, _):
        zb_v[i // 8, pl.ds((i % 8) * LANES, LANES)] = z16
        return 0
    lax.fori_loop(0, 8 * 8, body, 0)


def _zero_acc(acc, tile, zb_v):
    start = tile * (HROWS // NS)
    for j in range(HROWS // NS // 8):
        pltpu.sync_copy(zb_v, acc.at[pl.ds(start + j * 8, 8)])


def _get_cnt(cnt_hbm, r, cnt_v):
    pltpu.sync_copy(cnt_hbm.at[r], cnt_v)
    return cnt_v[pl.ds(0, LANES)][0]


def _prop_kernel(x_hbm, src_hbm, dst_hbm, val_hbm, cnt_hbm, out_hbm,
                 s_g, d_g, f_g, ss_v, sds_v, rows_v, zb_v, cnt_v, acc, sem):
    core = lax.axis_index("c")
    tile = lax.axis_index("s")
    _zero_zb(zb_v)
    r = core * NS + tile
    cnt = _get_cnt(cnt_hbm, r, cnt_v)
    pbase = r * PEPT
    iota = _iota16()
    for half in range(2):
        lo = half * HALF
        _zero_acc(acc, tile, zb_v)
        plsc.subcore_barrier()
        def grp(g, _):
            @pl.when(g * GRP < cnt)
            def _():
                pltpu.sync_copy(src_hbm.at[pl.ds(pbase + g * GRP, GRP)], s_g)
                pltpu.sync_copy(dst_hbm.at[pl.ds(pbase + g * GRP, GRP)], d_g)
                pltpu.sync_copy(val_hbm.at[pl.ds(pbase + g * GRP, GRP)], f_g)
                for l in range(8):
                    sl = pl.ds(l * LANES, LANES)
                    fl = f_g[sl] > 0
                    s16 = s_g[sl]
                    d16 = d_g[sl]
                    spread = (iota + l * LANES) & 63
                    ok = fl & (d16 >= lo) & (d16 < lo + HALF)
                    ss_v[sl] = jnp.where(fl, s16, spread)
                    sds_v[sl] = jnp.where(ok, d16 - lo, HALF + spread)
                pltpu.async_copy(x_hbm.at[ss_v], rows_v, sem).wait()
                pltpu.sync_copy(rows_v, acc.at[sds_v], add=True)
            return 0
        lax.fori_loop(0, NG, grp, 0)
        plsc.subcore_barrier()
        rt = HROWS // NS
        pltpu.sync_copy(
            acc.at[pl.ds(tile * rt, rt)],
            out_hbm.at[core, pl.ds(half * HROWS + tile * rt, rt)])
        plsc.subcore_barrier()


def _build_prop():
    return pl.kernel(
        _prop_kernel,
        out_type=jax.ShapeDtypeStruct((NC, 2 * HROWS, D), jnp.float32),
        mesh=_mesh,
        scratch_types=[
            pltpu.VMEM((GRP,), jnp.int32),
            pltpu.VMEM((GRP,), jnp.int32),
            pltpu.VMEM((GRP,), jnp.int32),
            pltpu.VMEM((GRP,), jnp.int32),
            pltpu.VMEM((GRP,), jnp.int32),
            pltpu.VMEM((GRP, D), jnp.float32),
            pltpu.VMEM((8, D), jnp.float32),
            pltpu.VMEM((LANES,), jnp.int32),
            pltpu.VMEM_SHARED((HROWS, D), jnp.float32),
            pltpu.SemaphoreType.DMA,
        ],
    )


def _attn_kernel(x_hbm, xq_hbm, x2_hbm, src_hbm, dst_hbm, val_hbm, cnt_hbm,
                 out_hbm, s_g, d_g, f_g, ss_v, sdg_v, sds_v,
                 xqr_v, xdr_v, x2r_v, ra_v, att16_v, idxr_v, za_v, zb_v,
                 cnt_v, acc, attn_sp, sem):
    core = lax.axis_index("c")
    tile = lax.axis_index("s")
    _zero_zb(zb_v)
    z16f = jnp.zeros((LANES,), jnp.float32)
    def za(i, _):
        za_v[pl.ds(i * LANES, LANES)] = z16f
        return 0
    lax.fori_loop(0, ASL // LANES, za, 0)
    abase = tile * ASL
    z16i = jnp.zeros((LANES,), jnp.int32)
    def ip(i, _):
        b = i // 8
        o = i % 8
        idxr_v[b, pl.ds(o * LANES, LANES)] = z16i + (abase + 8 * b + o)
        return 0
    lax.fori_loop(0, 8 * 8, ip, 0)
    r = core * NS + tile
    cnt = _get_cnt(cnt_hbm, r, cnt_v)
    pbase = r * PEPT
    iota = _iota16()
    for half in range(2):
        lo = half * HALF
        _zero_acc(acc, tile, zb_v)
        plsc.subcore_barrier()
        def grp(g, _):
            @pl.when(g * G2 < cnt)
            def _():
                pltpu.sync_copy(src_hbm.at[pl.ds(pbase + g * G2, G2)], s_g)
                pltpu.sync_copy(dst_hbm.at[pl.ds(pbase + g * G2, G2)], d_g)
                pltpu.sync_copy(val_hbm.at[pl.ds(pbase + g * G2, G2)], f_g)
                for l in range(4):
                    sl = pl.ds(l * LANES, LANES)
                    fl = f_g[sl] > 0
                    s16 = s_g[sl]
                    d16 = d_g[sl]
                    spread = (iota + l * LANES) & 63
                    ok = fl & (d16 >= lo) & (d16 < lo + HALF)
                    ss_v[sl] = jnp.where(fl, s16, spread)
                    sdg_v[sl] = jnp.where(fl, d16, spread)
                    sds_v[sl] = jnp.where(ok, d16 - lo, HALF + spread)
                cp1 = pltpu.async_copy(xq_hbm.at[ss_v], xqr_v, sem)
                cp2 = pltpu.async_copy(x_hbm.at[sdg_v], xdr_v, sem)
                cp3 = pltpu.async_copy(x2_hbm.at[ss_v], x2r_v, sem)
                cp1.wait()
                cp2.wait()
                cp3.wait()
                def prod(rr, _):
                    a16 = (xqr_v[rr, pl.ds(0, LANES)]
                           * xdr_v[rr, pl.ds(0, LANES)])
                    for l in range(1, 8):
                        a16 = a16 + (xqr_v[rr, pl.ds(l * LANES, LANES)]
                                     * xdr_v[rr, pl.ds(l * LANES, LANES)])
                    ra_v[pl.ds(rr * LANES, LANES)] = a16
                    return 0
                lax.fori_loop(0, G2, prod, 0)
                pltpu.sync_copy(za_v, attn_sp.at[pl.ds(abase, ASL)])
                def afire(b, _):
                    pltpu.async_copy(ra_v.at[pl.ds(b * GRP, GRP)],
                                     attn_sp.at[idxr_v.at[b]], sem, add=True)
                    return 0
                lax.fori_loop(0, 8, afire, 0)
                def adrain(b, _):
                    pltpu.make_async_copy(x_hbm.at[0], xdr_v.at[0], sem).wait()
                    return 0
                lax.fori_loop(0, 8, adrain, 0)
                def bfire(b, _):
                    pltpu.async_copy(attn_sp.at[idxr_v.at[b]],
                                     att16_v.at[pl.ds(b * GRP, GRP)], sem)
                    return 0
                lax.fori_loop(0, 8, bfire, 0)
                def bdrain(b, _):
                    pltpu.make_async_copy(x_hbm.at[0], xdr_v.at[0], sem).wait()
                    return 0
                lax.fori_loop(0, 8, bdrain, 0)
                def ymul(rr, _):
                    at16 = att16_v[pl.ds(rr * LANES, LANES)]
                    for l in range(8):
                        xdr_v[rr, pl.ds(l * LANES, LANES)] = (
                            at16 * x2r_v[rr, pl.ds(l * LANES, LANES)])
                    return 0
                lax.fori_loop(0, G2, ymul, 0)
                pltpu.sync_copy(xdr_v, acc.at[sds_v], add=True)
            return 0
        lax.fori_loop(0, NG2, grp, 0)
        plsc.subcore_barrier()
        rt = HROWS // NS
        pltpu.sync_copy(
            acc.at[pl.ds(tile * rt, rt)],
            out_hbm.at[core, pl.ds(half * HROWS + tile * rt, rt)])
        plsc.subcore_barrier()


def _build_attn():
    return pl.kernel(
        _attn_kernel,
        out_type=jax.ShapeDtypeStruct((NC, 2 * HROWS, D), jnp.float32),
        mesh=_mesh,
        scratch_types=[
            pltpu.VMEM((G2,), jnp.int32),
            pltpu.VMEM((G2,), jnp.int32),
            pltpu.VMEM((G2,), jnp.int32),
            pltpu.VMEM((G2,), jnp.int32),
            pltpu.VMEM((G2,), jnp.int32),
            pltpu.VMEM((G2,), jnp.int32),
            pltpu.VMEM((G2, D), jnp.float32),
            pltpu.VMEM((G2, D), jnp.float32),
            pltpu.VMEM((G2, D), jnp.float32),
            pltpu.VMEM((G2 * LANES,), jnp.float32),
            pltpu.VMEM((G2 * LANES,), jnp.float32),
            pltpu.VMEM((8, GRP), jnp.int32),
            pltpu.VMEM((ASL,), jnp.float32),
            pltpu.VMEM((8, D), jnp.float32),
            pltpu.VMEM((LANES,), jnp.int32),
            pltpu.VMEM_SHARED((HROWS, D), jnp.float32),
            pltpu.VMEM_SHARED((NS * ASL,), jnp.float32),
            pltpu.SemaphoreType.DMA,
        ],
    )


_RB = 1000


def _t1_body(x_ref, p0_ref, p1_ref, w1_ref, q_ref, w2_ref,
             x1_ref, xq_ref, x2_ref):
    x = x_ref[...]
    p = p0_ref[...] + p1_ref[...]
    w1 = w1_ref[...]
    q = q_ref[...]
    h = jnp.dot(p, w1.T, preferred_element_type=jnp.float32)
    x1_ref[...] = x + jnp.dot(h, q, preferred_element_type=jnp.float32) * (
        float(N) / float(E))
    xq_ref[...] = jnp.dot(x, q, preferred_element_type=jnp.float32)
    x2_ref[...] = jnp.dot(x, w2_ref[...].T, preferred_element_type=jnp.float32)


def _t1(x, p0, p1, w1, q, w2):
    full = pl.BlockSpec((D, D), lambda i: (0, 0))
    blk = pl.BlockSpec((_RB, D), lambda i: (i, 0))
    return pl.pallas_call(
        _t1_body,
        grid=(N // _RB,),
        in_specs=[blk, blk, blk, full, full, full],
        out_specs=[blk, blk, blk],
        out_shape=[jax.ShapeDtypeStruct((N, D), jnp.float32)] * 3,
    )(x, p0, p1, w1, q, w2)


def _t2_body(x1_ref, a0_ref, a1_ref, q_ref, cnt_ref, out_ref):
    a = a0_ref[...] + a1_ref[...]
    cnt2 = jnp.sum(cnt_ref[...][:, 0]).astype(jnp.float32)
    scale = float(N) / cnt2
    out_ref[...] = x1_ref[...] - jnp.dot(
        a, q_ref[...], preferred_element_type=jnp.float32) * scale


def _t2(x1, a0, a1, q, cnt2):
    full = pl.BlockSpec((D, D), lambda i: (0, 0))
    blk = pl.BlockSpec((_RB, D), lambda i: (i, 0))
    cblk = pl.BlockSpec((NT, LANES), lambda i: (0, 0))
    return pl.pallas_call(
        _t2_body,
        grid=(N // _RB,),
        in_specs=[blk, blk, blk, full, cblk],
        out_specs=blk,
        out_shape=jax.ShapeDtypeStruct((N, D), jnp.float32),
    )(x1, a0, a1, q, cnt2)


def _halves(arr):
    return (arr[:, :HALF], arr[:, HROWS:HROWS + HALF])


def kernel(input, edge_index, edge_index_2, Q, mask, W1_0, W2_0, W1_1, W2_1):
    t = jnp.full((TSZ,), -1, jnp.int32)
    mask_f = mask.reshape(-1)
    e1_f = edge_index.reshape(-1)
    e2_f = edge_index_2.reshape(-1)
    tok1 = _build_e1()(t, mask_f)
    tok2 = _build_claim(0)(t, e1_f, tok1)
    src1, dst1, val1, cnt1, _tc1 = _build_compact(0)(t, e1_f, tok2)
    tok4 = _build_claim(E)(t, e2_f, cnt1)
    src2, dst2, val2, cnt2, tcnt2 = _build_compact(E)(t, e2_f, tok4)

    prop_k = _build_prop()
    attn_k = _build_attn()

    x = input
    for (w1, w2) in ((W1_0, W2_0), (W1_1, W2_1)):
        prop = prop_k(x, src1, dst1, val1, cnt1)
        pa, pb = _halves(prop)
        p0 = jnp.concatenate([pa[0], pb[0]], axis=0)
        p1 = jnp.concatenate([pa[1], pb[1]], axis=0)
        x1, xq, x2 = _t1(x, p0, p1, w1, Q, w2)
        agg = attn_k(x, xq, x2, src2, dst2, val2, cnt2)
        aa, ab = _halves(agg)
        a0 = jnp.concatenate([aa[0], ab[0]], axis=0)
        a1 = jnp.concatenate([aa[1], ab[1]], axis=0)
        x = _t2(x1, a0, a1, Q, tcnt2)
    return x


# speedup vs baseline: 16.9326x; 1.1157x over previous
import functools

import jax
import jax.numpy as jnp
from jax import lax
from jax.experimental import pallas as pl
from jax.experimental.pallas import tpu as pltpu
from jax.experimental.pallas import tpu_sc as plsc

NC = 2
NS = 16
NT = NC * NS
LANES = 16

N = 10000
D = 128
E = 320000
EPT = E // NT
NSL = EPT // LANES
GRP = 128
NG = (EPT + GRP - 1) // GRP
DUMP = N * N
TSZ = N * N + NT * GRP + 64
PEPT = NG * GRP
CSL = 656
TOTSLOT = 100
HROWS = 10240
G2 = 64
NG2 = PEPT // G2
ASL = 160

_mesh = plsc.VectorSubcoreMesh(core_axis_name="c", subcore_axis_name="s")


def _wid():
    return lax.axis_index("c") * NS + lax.axis_index("s")


def _iota16():
    return lax.broadcasted_iota(jnp.int32, (LANES,), 0)


def _dump16(wid, l):
    return DUMP + wid * GRP + l * LANES + _iota16()


def _compute_keys(wid, s_v, d_v, k_v):
    def body(i, _):
        g = i // 8
        o = (i % 8) * LANES
        s16 = s_v[pl.ds(i * LANES, LANES)]
        d16 = d_v[pl.ds(i * LANES, LANES)]
        k_v[g, pl.ds(o, LANES)] = s16 * N + d16
        return 0
    lax.fori_loop(0, NSL, body, 0)
    for l in range(1, 8):
        k_v[NG - 1, pl.ds(l * LANES, LANES)] = _dump16(wid, l)


def _load_pair(pair_hbm, base, s_v, d_v):
    pltpu.sync_copy(pair_hbm.at[pl.ds(base, EPT)], s_v)
    pltpu.sync_copy(pair_hbm.at[pl.ds(E + base, EPT)], d_v)


def _gather_rows(t_hbm, k_v, o_v, sem):
    def fire(g, _):
        pltpu.async_copy(t_hbm.at[k_v.at[g]], o_v.at[g], sem)
        return 0
    lax.fori_loop(0, NG, fire, 0)
    def drain(g, _):
        pltpu.make_async_copy(t_hbm.at[pl.ds(0, GRP)], o_v.at[g], sem).wait()
        return 0
    lax.fori_loop(0, NG, drain, 0)


def _mask_kernel(t_hbm, mask_hbm, tok_out, s_v, d_v, k_v, val_v, sem):
    wid = _wid()
    base = wid * EPT
    _load_pair(mask_hbm, base, s_v, d_v)
    _compute_keys(wid, s_v, d_v, k_v)
    m2 = jnp.full((LANES,), -2, jnp.int32)
    for l in range(8):
        val_v[pl.ds(l * LANES, LANES)] = m2
    def fire(g, _):
        pltpu.async_copy(val_v, t_hbm.at[k_v.at[g]], sem)
        return 0
    lax.fori_loop(0, NG, fire, 0)
    def drain(g, _):
        pltpu.make_async_copy(t_hbm.at[pl.ds(0, GRP)], k_v.at[g], sem).wait()
        return 0
    lax.fori_loop(0, NG, drain, 0)
    @pl.when(_wid() == 0)
    def _():
        val_v[pl.ds(0, LANES)] = jnp.zeros((LANES,), jnp.int32)
        pltpu.sync_copy(val_v.at[pl.ds(0, LANES)], tok_out)


def _build_e1():
    return pl.kernel(
        _mask_kernel,
        out_type=jax.ShapeDtypeStruct((LANES,), jnp.int32),
        mesh=_mesh,
        scratch_types=[
            pltpu.VMEM((EPT,), jnp.int32),
            pltpu.VMEM((EPT,), jnp.int32),
            pltpu.VMEM((NG, GRP), jnp.int32),
            pltpu.VMEM((GRP,), jnp.int32),
            pltpu.SemaphoreType.DMA,
        ],
    )


def _claim_kernel(id_base, t_hbm, ei_hbm, tok_in, tok_out,
                  s_v, d_v, k_v, m_v, q_v, id_v, sem):
    del tok_in
    wid = _wid()
    base = wid * EPT
    _load_pair(ei_hbm, base, s_v, d_v)
    _compute_keys(wid, s_v, d_v, k_v)
    _gather_rows(t_hbm, k_v, m_v, sem)
    iota = _iota16()
    def sel(i, _):
        g = i // 8
        o = (i % 8) * LANES
        m16 = m_v[g, pl.ds(o, LANES)]
        k16 = k_v[g, pl.ds(o, LANES)]
        q_v[g, pl.ds(o, LANES)] = jnp.where(
            m16 != -1, k16, DUMP + wid * GRP + o + iota)
        p = i * LANES + iota
        id_v[g, pl.ds(o, LANES)] = id_base + base + p
        return 0
    lax.fori_loop(0, NSL, sel, 0)
    m4 = jnp.full((LANES,), -4, jnp.int32)
    for l in range(1, 8):
        q_v[NG - 1, pl.ds(l * LANES, LANES)] = _dump16(wid, l)
        id_v[NG - 1, pl.ds(l * LANES, LANES)] = m4
    def fire2(g, _):
        pltpu.async_copy(id_v.at[g], t_hbm.at[q_v.at[g]], sem)
        return 0
    lax.fori_loop(0, NG, fire2, 0)
    def drain2(g, _):
        pltpu.make_async_copy(t_hbm.at[pl.ds(0, GRP)], m_v.at[g], sem).wait()
        return 0
    lax.fori_loop(0, NG, drain2, 0)
    @pl.when(wid == 0)
    def _():
        id_v[0, pl.ds(0, LANES)] = jnp.zeros((LANES,), jnp.int32)
        pltpu.sync_copy(id_v.at[0, pl.ds(0, LANES)], tok_out)


def _build_claim(id_base):
    return pl.kernel(
        functools.partial(_claim_kernel, id_base),
        out_type=jax.ShapeDtypeStruct((LANES,), jnp.int32),
        mesh=_mesh,
        scratch_types=[
            pltpu.VMEM((EPT,), jnp.int32),
            pltpu.VMEM((EPT,), jnp.int32),
            pltpu.VMEM((NG, GRP), jnp.int32),
            pltpu.VMEM((NG, GRP), jnp.int32),
            pltpu.VMEM((NG, GRP), jnp.int32),
            pltpu.VMEM((NG, GRP), jnp.int32),
            pltpu.SemaphoreType.DMA,
        ],
    )


def _compact_kernel(id_base, t_hbm, ei_hbm, tok_in, src_out, dst_out, val_out,
                    cnt_out, tcnt_out, s_v, d_v, k_v, o_v, f_v, sidx_v,
                    cs_v, cd_v, cf_v, zc_v, gcnt_v, cnt_v, cnts_sp, sem):
    del tok_in
    wid = _wid()
    tile = lax.axis_index("s")
    base = wid * EPT
    _load_pair(ei_hbm, base, s_v, d_v)
    _compute_keys(wid, s_v, d_v, k_v)
    _gather_rows(t_hbm, k_v, o_v, sem)
    z16 = jnp.zeros((LANES,), jnp.int32)
    def zc(i, _):
        zc_v[pl.ds(i * LANES, LANES)] = z16
        return 0
    lax.fori_loop(0, CSL // LANES, zc, 0)
    cbase = tile * CSL
    pltpu.sync_copy(zc_v, cnts_sp.at[pl.ds(cbase, CSL)])
    iota = _iota16()
    def flg(i, _):
        g = i // 8
        o = (i % 8) * LANES
        o16 = o_v[g, pl.ds(o, LANES)]
        won = o16 == (id_base + base + i * LANES + iota)
        f_v[g, pl.ds(o, LANES)] = jnp.where(won, 1, 0)
        sidx_v[g, pl.ds(o, LANES)] = z16 + (cbase + g)
        return 0
    lax.fori_loop(0, NSL, flg, 0)
    for l in range(1, 8):
        f_v[NG - 1, pl.ds(l * LANES, LANES)] = z16
        sidx_v[NG - 1, pl.ds(l * LANES, LANES)] = z16 + (cbase + NG - 1)
    def cfire(g, _):
        pltpu.async_copy(f_v.at[g], cnts_sp.at[sidx_v.at[g]], sem, add=True)
        return 0
    lax.fori_loop(0, NG, cfire, 0)
    def cdrain(g, _):
        pltpu.make_async_copy(t_hbm.at[pl.ds(0, GRP)], k_v.at[g], sem).wait()
        return 0
    lax.fori_loop(0, NG, cdrain, 0)
    pltpu.sync_copy(cnts_sp.at[pl.ds(cbase, GRP)], gcnt_v)
    iota = _iota16()
    for l in range(8):
        o = l * LANES
        gv = gcnt_v[pl.ds(o, LANES)]
        o_v[0, pl.ds(o, LANES)] = jnp.where(o + iota < NG, gv, 0)
        sidx_v[0, pl.ds(o, LANES)] = z16 + (cbase + TOTSLOT)
    pltpu.sync_copy(o_v.at[0], cnts_sp.at[sidx_v.at[0]], add=True)
    pltpu.sync_copy(cnts_sp.at[pl.ds(cbase, GRP)], gcnt_v)
    gslices = [gcnt_v[pl.ds(k * LANES, LANES)] for k in range(8)]
    pos = jnp.int32(0)
    for g in range(NG):
        gc = gslices[g // LANES][g % LANES]
        @pl.when(gc > 0)
        def _(g=g, pos=pos):
            for l in range(8):
                o = l * LANES
                cs_v[pl.ds(pos * GRP + o, LANES)] = (
                    s_v[pl.ds(g * GRP + o, LANES)]
                    if g * GRP + o + LANES <= EPT
                    else z16)
                cd_v[pl.ds(pos * GRP + o, LANES)] = (
                    d_v[pl.ds(g * GRP + o, LANES)]
                    if g * GRP + o + LANES <= EPT
                    else z16)
                cf_v[pl.ds(pos * GRP + o, LANES)] = f_v[g, pl.ds(o, LANES)]
        pos = pos + jnp.where(gc > 0, 1, 0)
    pbase = wid * PEPT
    pltpu.sync_copy(cs_v.at[pl.ds(0, PEPT)], src_out.at[pl.ds(pbase, PEPT)])
    pltpu.sync_copy(cd_v.at[pl.ds(0, PEPT)], dst_out.at[pl.ds(pbase, PEPT)])
    pltpu.sync_copy(cf_v.at[pl.ds(0, PEPT)], val_out.at[pl.ds(pbase, PEPT)])
    cnt_v[pl.ds(0, LANES)] = z16 + pos * GRP
    pltpu.sync_copy(cnt_v, cnt_out.at[wid])
    tslice = gcnt_v[pl.ds((TOTSLOT // LANES) * LANES, LANES)]
    cnt_v[pl.ds(0, LANES)] = z16 + tslice[TOTSLOT % LANES]
    pltpu.sync_copy(cnt_v, tcnt_out.at[wid])


def _build_compact(id_base):
    return pl.kernel(
        functools.partial(_compact_kernel, id_base),
        out_type=(
            jax.ShapeDtypeStruct((NT * PEPT,), jnp.int32),
            jax.ShapeDtypeStruct((NT * PEPT,), jnp.int32),
            jax.ShapeDtypeStruct((NT * PEPT,), jnp.int32),
            jax.ShapeDtypeStruct((NT, LANES), jnp.int32),
            jax.ShapeDtypeStruct((NT, LANES), jnp.int32),
        ),
        mesh=_mesh,
        scratch_types=[
            pltpu.VMEM((EPT,), jnp.int32),
            pltpu.VMEM((EPT,), jnp.int32),
            pltpu.VMEM((NG, GRP), jnp.int32),
            pltpu.VMEM((NG, GRP), jnp.int32),
            pltpu.VMEM((NG, GRP), jnp.int32),
            pltpu.VMEM((NG, GRP), jnp.int32),
            pltpu.VMEM((PEPT,), jnp.int32),
            pltpu.VMEM((PEPT,), jnp.int32),
            pltpu.VMEM((PEPT,), jnp.int32),
            pltpu.VMEM((CSL,), jnp.int32),
            pltpu.VMEM((GRP,), jnp.int32),
            pltpu.VMEM((LANES,), jnp.int32),
            pltpu.VMEM_SHARED((NS * CSL,), jnp.int32),
            pltpu.SemaphoreType.DMA,
        ],
    )


def _zero_zb(zb_v):
    z16 = jnp.zeros((LANES,), jnp.float32)
    def body(i, _):
        zb_v[i // 8, pl.ds((i % 8) * LANES, LANES)] = z16
        return 0
    lax.fori_loop(0, 8 * 8, body, 0)


def _zero_acc(acc, tile, zb_v):
    start = tile * (HROWS // NS)
    for j in range(HROWS // NS // 8):
        pltpu.sync_copy(zb_v, acc.at[pl.ds(start + j * 8, 8)])


def _get_cnt(cnt_hbm, r, cnt_v):
    pltpu.sync_copy(cnt_hbm.at[r], cnt_v)
    return cnt_v[pl.ds(0, LANES)][0]


def _prop_kernel(x_hbm, src_hbm, dst_hbm, val_hbm, cnt_hbm, out_hbm,
                 s_g, d_g, f_g, ss_v, sds_v, rows_v, zb_v, cnt_v, acc, sem):
    core = lax.axis_index("c")
    tile = lax.axis_index("s")
    _zero_zb(zb_v)
    r = core * NS + tile
    cnt = _get_cnt(cnt_hbm, r, cnt_v)
    pbase = r * PEPT
    iota = _iota16()
    _zero_acc(acc, tile, zb_v)
    plsc.subcore_barrier()
    def grp(g, _):
        @pl.when(g * GRP < cnt)
        def _():
            pltpu.sync_copy(src_hbm.at[pl.ds(pbase + g * GRP, GRP)], s_g)
            pltpu.sync_copy(dst_hbm.at[pl.ds(pbase + g * GRP, GRP)], d_g)
            pltpu.sync_copy(val_hbm.at[pl.ds(pbase + g * GRP, GRP)], f_g)
            for l in range(8):
                sl = pl.ds(l * LANES, LANES)
                fl = f_g[sl] > 0
                s16 = s_g[sl]
                d16 = d_g[sl]
                spread = (iota + l * LANES) & 63
                ss_v[sl] = jnp.where(fl, s16, spread)
                sds_v[sl] = jnp.where(fl, d16, N + spread)
            pltpu.async_copy(x_hbm.at[ss_v], rows_v, sem).wait()
            pltpu.sync_copy(rows_v, acc.at[sds_v], add=True)
        return 0
    lax.fori_loop(0, NG, grp, 0)
    plsc.subcore_barrier()
    rt = HROWS // NS
    pltpu.sync_copy(acc.at[pl.ds(tile * rt, rt)],
                    out_hbm.at[core, pl.ds(tile * rt, rt)])


def _build_prop():
    return pl.kernel(
        _prop_kernel,
        out_type=jax.ShapeDtypeStruct((NC, HROWS, D), jnp.float32),
        mesh=_mesh,
        scratch_types=[
            pltpu.VMEM((GRP,), jnp.int32),
            pltpu.VMEM((GRP,), jnp.int32),
            pltpu.VMEM((GRP,), jnp.int32),
            pltpu.VMEM((GRP,), jnp.int32),
            pltpu.VMEM((GRP,), jnp.int32),
            pltpu.VMEM((GRP, D), jnp.float32),
            pltpu.VMEM((8, D), jnp.float32),
            pltpu.VMEM((LANES,), jnp.int32),
            pltpu.VMEM_SHARED((HROWS, D), jnp.float32),
            pltpu.SemaphoreType.DMA,
        ],
    )


def _attn_kernel(x_hbm, xq_hbm, x2_hbm, src_hbm, dst_hbm, val_hbm, cnt_hbm,
                 out_hbm, s_g, d_g, f_g, ss_v, sdg_v, sds_v,
                 xqr_v, xdr_v, x2r_v, ra_v, att16_v, idxr_v, za_v, zb_v,
                 cnt_v, acc, attn_sp, sem):
    core = lax.axis_index("c")
    tile = lax.axis_index("s")
    _zero_zb(zb_v)
    z16f = jnp.zeros((LANES,), jnp.float32)
    def za(i, _):
        za_v[pl.ds(i * LANES, LANES)] = z16f
        return 0
    lax.fori_loop(0, ASL // LANES, za, 0)
    abase = tile * ASL
    z16i = jnp.zeros((LANES,), jnp.int32)
    def ip(i, _):
        b = i // 8
        o = i % 8
        idxr_v[b, pl.ds(o * LANES, LANES)] = z16i + (abase + 8 * b + o)
        return 0
    lax.fori_loop(0, 8 * 8, ip, 0)
    r = core * NS + tile
    cnt = _get_cnt(cnt_hbm, r, cnt_v)
    pbase = r * PEPT
    iota = _iota16()
    if True:
        _zero_acc(acc, tile, zb_v)
        plsc.subcore_barrier()
        def grp(g, _):
            @pl.when(g * G2 < cnt)
            def _():
                pltpu.sync_copy(src_hbm.at[pl.ds(pbase + g * G2, G2)], s_g)
                pltpu.sync_copy(dst_hbm.at[pl.ds(pbase + g * G2, G2)], d_g)
                pltpu.sync_copy(val_hbm.at[pl.ds(pbase + g * G2, G2)], f_g)
                for l in range(4):
                    sl = pl.ds(l * LANES, LANES)
                    fl = f_g[sl] > 0
                    s16 = s_g[sl]
                    d16 = d_g[sl]
                    spread = (iota + l * LANES) & 63
                    ss_v[sl] = jnp.where(fl, s16, spread)
                    sdg_v[sl] = jnp.where(fl, d16, spread)
                    sds_v[sl] = jnp.where(fl, d16, N + spread)
                cp1 = pltpu.async_copy(xq_hbm.at[ss_v], xqr_v, sem)
                cp2 = pltpu.async_copy(x_hbm.at[sdg_v], xdr_v, sem)
                cp3 = pltpu.async_copy(x2_hbm.at[ss_v], x2r_v, sem)
                cp1.wait()
                cp2.wait()
                cp3.wait()
                def prod(rr, _):
                    a16 = (xqr_v[rr, pl.ds(0, LANES)]
                           * xdr_v[rr, pl.ds(0, LANES)])
                    for l in range(1, 8):
                        a16 = a16 + (xqr_v[rr, pl.ds(l * LANES, LANES)]
                                     * xdr_v[rr, pl.ds(l * LANES, LANES)])
                    ra_v[pl.ds(rr * LANES, LANES)] = a16
                    return 0
                lax.fori_loop(0, G2, prod, 0)
                pltpu.sync_copy(za_v, attn_sp.at[pl.ds(abase, ASL)])
                def afire(b, _):
                    pltpu.async_copy(ra_v.at[pl.ds(b * GRP, GRP)],
                                     attn_sp.at[idxr_v.at[b]], sem, add=True)
                    return 0
                lax.fori_loop(0, 8, afire, 0)
                def adrain(b, _):
                    pltpu.make_async_copy(x_hbm.at[0], xdr_v.at[0], sem).wait()
                    return 0
                lax.fori_loop(0, 8, adrain, 0)
                def bfire(b, _):
                    pltpu.async_copy(attn_sp.at[idxr_v.at[b]],
                                     att16_v.at[pl.ds(b * GRP, GRP)], sem)
                    return 0
                lax.fori_loop(0, 8, bfire, 0)
                def bdrain(b, _):
                    pltpu.make_async_copy(x_hbm.at[0], xdr_v.at[0], sem).wait()
                    return 0
                lax.fori_loop(0, 8, bdrain, 0)
                def ymul(rr, _):
                    at16 = att16_v[pl.ds(rr * LANES, LANES)]
                    for l in range(8):
                        xdr_v[rr, pl.ds(l * LANES, LANES)] = (
                            at16 * x2r_v[rr, pl.ds(l * LANES, LANES)])
                    return 0
                lax.fori_loop(0, G2, ymul, 0)
                pltpu.sync_copy(xdr_v, acc.at[sds_v], add=True)
            return 0
        lax.fori_loop(0, NG2, grp, 0)
        plsc.subcore_barrier()
        rt = HROWS // NS
        pltpu.sync_copy(acc.at[pl.ds(tile * rt, rt)],
                        out_hbm.at[core, pl.ds(tile * rt, rt)])


def _build_attn():
    return pl.kernel(
        _attn_kernel,
        out_type=jax.ShapeDtypeStruct((NC, HROWS, D), jnp.float32),
        mesh=_mesh,
        scratch_types=[
            pltpu.VMEM((G2,), jnp.int32),
            pltpu.VMEM((G2,), jnp.int32),
            pltpu.VMEM((G2,), jnp.int32),
            pltpu.VMEM((G2,), jnp.int32),
            pltpu.VMEM((G2,), jnp.int32),
            pltpu.VMEM((G2,), jnp.int32),
            pltpu.VMEM((G2, D), jnp.float32),
            pltpu.VMEM((G2, D), jnp.float32),
            pltpu.VMEM((G2, D), jnp.float32),
            pltpu.VMEM((G2 * LANES,), jnp.float32),
            pltpu.VMEM((G2 * LANES,), jnp.float32),
            pltpu.VMEM((8, GRP), jnp.int32),
            pltpu.VMEM((ASL,), jnp.float32),
            pltpu.VMEM((8, D), jnp.float32),
            pltpu.VMEM((LANES,), jnp.int32),
            pltpu.VMEM_SHARED((HROWS, D), jnp.float32),
            pltpu.VMEM_SHARED((NS * ASL,), jnp.float32),
            pltpu.SemaphoreType.DMA,
        ],
    )


_RB = 1000


def _t1_body(x_ref, p0_ref, p1_ref, w1_ref, q_ref, w2_ref,
             x1_ref, xq_ref, x2_ref):
    x = x_ref[...]
    p = p0_ref[...] + p1_ref[...]
    w1 = w1_ref[...]
    q = q_ref[...]
    h = jnp.dot(p, w1.T, preferred_element_type=jnp.float32)
    x1_ref[...] = x + jnp.dot(h, q, preferred_element_type=jnp.float32) * (
        float(N) / float(E))
    xq_ref[...] = jnp.dot(x, q, preferred_element_type=jnp.float32)
    x2_ref[...] = jnp.dot(x, w2_ref[...].T, preferred_element_type=jnp.float32)


def _t1(x, p0, p1, w1, q, w2):
    full = pl.BlockSpec((D, D), lambda i: (0, 0))
    blk = pl.BlockSpec((_RB, D), lambda i: (i, 0))
    return pl.pallas_call(
        _t1_body,
        grid=(N // _RB,),
        in_specs=[blk, blk, blk, full, full, full],
        out_specs=[blk, blk, blk],
        out_shape=[jax.ShapeDtypeStruct((N, D), jnp.float32)] * 3,
    )(x, p0, p1, w1, q, w2)


def _t2_body(x1_ref, a0_ref, a1_ref, q_ref, cnt_ref, out_ref):
    a = a0_ref[...] + a1_ref[...]
    cnt2 = jnp.sum(cnt_ref[...][:, 0]).astype(jnp.float32)
    scale = float(N) / cnt2
    out_ref[...] = x1_ref[...] - jnp.dot(
        a, q_ref[...], preferred_element_type=jnp.float32) * scale


def _t2(x1, a0, a1, q, cnt2):
    full = pl.BlockSpec((D, D), lambda i: (0, 0))
    blk = pl.BlockSpec((_RB, D), lambda i: (i, 0))
    cblk = pl.BlockSpec((NT, LANES), lambda i: (0, 0))
    return pl.pallas_call(
        _t2_body,
        grid=(N // _RB,),
        in_specs=[blk, blk, blk, full, cblk],
        out_specs=blk,
        out_shape=jax.ShapeDtypeStruct((N, D), jnp.float32),
    )(x1, a0, a1, q, cnt2)


def kernel(input, edge_index, edge_index_2, Q, mask, W1_0, W2_0, W1_1, W2_1):
    t = jnp.full((TSZ,), -1, jnp.int32)
    mask_f = mask.reshape(-1)
    e1_f = edge_index.reshape(-1)
    e2_f = edge_index_2.reshape(-1)
    tok1 = _build_e1()(t, mask_f)
    tok2 = _build_claim(0)(t, e1_f, tok1)
    src1, dst1, val1, cnt1, _tc1 = _build_compact(0)(t, e1_f, tok2)
    tok4 = _build_claim(E)(t, e2_f, cnt1)
    src2, dst2, val2, cnt2, tcnt2 = _build_compact(E)(t, e2_f, tok4)

    prop_k = _build_prop()
    attn_k = _build_attn()

    x = input
    for (w1, w2) in ((W1_0, W2_0), (W1_1, W2_1)):
        prop = prop_k(x, src1, dst1, val1, cnt1)
        x1, xq, x2 = _t1(x, prop[0, :N], prop[1, :N], w1, Q, w2)
        agg = attn_k(x, xq, x2, src2, dst2, val2, cnt2)
        x = _t2(x1, agg[0, :N], agg[1, :N], Q, tcnt2)
    return x


# speedup vs baseline: 17.0236x; 1.0054x over previous
import functools

import jax
import jax.numpy as jnp
from jax import lax
from jax.experimental import pallas as pl
from jax.experimental.pallas import tpu as pltpu
from jax.experimental.pallas import tpu_sc as plsc

NC = 2
NS = 16
NT = NC * NS
LANES = 16

N = 10000
D = 128
E = 320000
EPT = E // NT
NSL = EPT // LANES
GRP = 128
NG = (EPT + GRP - 1) // GRP
DUMP = N * N
TSZ = N * N + NT * GRP + 64
PEPT = NG * GRP
CSL = 656
TOTSLOT = 100
HROWS = 10240
G2 = 64
NG2 = PEPT // G2
ASL = 160

_mesh = plsc.VectorSubcoreMesh(core_axis_name="c", subcore_axis_name="s")


def _wid():
    return lax.axis_index("c") * NS + lax.axis_index("s")


def _iota16():
    return lax.broadcasted_iota(jnp.int32, (LANES,), 0)


def _dump16(wid, l):
    return DUMP + wid * GRP + l * LANES + _iota16()


def _compute_keys(wid, s_v, d_v, k_v):
    def body(i, _):
        g = i // 8
        o = (i % 8) * LANES
        s16 = s_v[pl.ds(i * LANES, LANES)]
        d16 = d_v[pl.ds(i * LANES, LANES)]
        k_v[g, pl.ds(o, LANES)] = s16 * N + d16
        return 0
    lax.fori_loop(0, NSL, body, 0)
    for l in range(1, 8):
        k_v[NG - 1, pl.ds(l * LANES, LANES)] = _dump16(wid, l)


def _load_pair(pair_hbm, base, s_v, d_v):
    pltpu.sync_copy(pair_hbm.at[pl.ds(base, EPT)], s_v)
    pltpu.sync_copy(pair_hbm.at[pl.ds(E + base, EPT)], d_v)


def _gather_rows(t_hbm, k_v, o_v, sem):
    def fire(g, _):
        pltpu.async_copy(t_hbm.at[k_v.at[g]], o_v.at[g], sem)
        return 0
    lax.fori_loop(0, NG, fire, 0)
    def drain(g, _):
        pltpu.make_async_copy(t_hbm.at[pl.ds(0, GRP)], o_v.at[g], sem).wait()
        return 0
    lax.fori_loop(0, NG, drain, 0)


def _mask_kernel(t_hbm, mask_hbm, tok_out, s_v, d_v, k_v, val_v, sem):
    wid = _wid()
    base = wid * EPT
    _load_pair(mask_hbm, base, s_v, d_v)
    _compute_keys(wid, s_v, d_v, k_v)
    m2 = jnp.full((LANES,), -2, jnp.int32)
    for l in range(8):
        val_v[pl.ds(l * LANES, LANES)] = m2
    def fire(g, _):
        pltpu.async_copy(val_v, t_hbm.at[k_v.at[g]], sem)
        return 0
    lax.fori_loop(0, NG, fire, 0)
    def drain(g, _):
        pltpu.make_async_copy(t_hbm.at[pl.ds(0, GRP)], k_v.at[g], sem).wait()
        return 0
    lax.fori_loop(0, NG, drain, 0)
    @pl.when(_wid() == 0)
    def _():
        val_v[pl.ds(0, LANES)] = jnp.zeros((LANES,), jnp.int32)
        pltpu.sync_copy(val_v.at[pl.ds(0, LANES)], tok_out)


def _build_e1():
    return pl.kernel(
        _mask_kernel,
        out_type=jax.ShapeDtypeStruct((LANES,), jnp.int32),
        mesh=_mesh,
        scratch_types=[
            pltpu.VMEM((EPT,), jnp.int32),
            pltpu.VMEM((EPT,), jnp.int32),
            pltpu.VMEM((NG, GRP), jnp.int32),
            pltpu.VMEM((GRP,), jnp.int32),
            pltpu.SemaphoreType.DMA,
        ],
    )


def _claim_kernel(id_base, t_hbm, ei_hbm, tok_in, tok_out,
                  s_v, d_v, k_v, m_v, q_v, id_v, sem):
    del tok_in
    wid = _wid()
    base = wid * EPT
    _load_pair(ei_hbm, base, s_v, d_v)
    _compute_keys(wid, s_v, d_v, k_v)
    _gather_rows(t_hbm, k_v, m_v, sem)
    iota = _iota16()
    def sel(i, _):
        g = i // 8
        o = (i % 8) * LANES
        m16 = m_v[g, pl.ds(o, LANES)]
        k16 = k_v[g, pl.ds(o, LANES)]
        q_v[g, pl.ds(o, LANES)] = jnp.where(
            m16 != -1, k16, DUMP + wid * GRP + o + iota)
        p = i * LANES + iota
        id_v[g, pl.ds(o, LANES)] = id_base + base + p
        return 0
    lax.fori_loop(0, NSL, sel, 0)
    m4 = jnp.full((LANES,), -4, jnp.int32)
    for l in range(1, 8):
        q_v[NG - 1, pl.ds(l * LANES, LANES)] = _dump16(wid, l)
        id_v[NG - 1, pl.ds(l * LANES, LANES)] = m4
    def fire2(g, _):
        pltpu.async_copy(id_v.at[g], t_hbm.at[q_v.at[g]], sem)
        return 0
    lax.fori_loop(0, NG, fire2, 0)
    def drain2(g, _):
        pltpu.make_async_copy(t_hbm.at[pl.ds(0, GRP)], m_v.at[g], sem).wait()
        return 0
    lax.fori_loop(0, NG, drain2, 0)
    @pl.when(wid == 0)
    def _():
        id_v[0, pl.ds(0, LANES)] = jnp.zeros((LANES,), jnp.int32)
        pltpu.sync_copy(id_v.at[0, pl.ds(0, LANES)], tok_out)


def _build_claim(id_base):
    return pl.kernel(
        functools.partial(_claim_kernel, id_base),
        out_type=jax.ShapeDtypeStruct((LANES,), jnp.int32),
        mesh=_mesh,
        scratch_types=[
            pltpu.VMEM((EPT,), jnp.int32),
            pltpu.VMEM((EPT,), jnp.int32),
            pltpu.VMEM((NG, GRP), jnp.int32),
            pltpu.VMEM((NG, GRP), jnp.int32),
            pltpu.VMEM((NG, GRP), jnp.int32),
            pltpu.VMEM((NG, GRP), jnp.int32),
            pltpu.SemaphoreType.DMA,
        ],
    )


def _compact_kernel(id_base, t_hbm, ei_hbm, tok_in, src_out, dst_out, val_out,
                    cnt_out, tcnt_out, s_v, d_v, k_v, o_v, f_v, sidx_v,
                    cs_v, cd_v, cf_v, zc_v, gcnt_v, cnt_v, cnts_sp, sem):
    del tok_in
    wid = _wid()
    tile = lax.axis_index("s")
    base = wid * EPT
    _load_pair(ei_hbm, base, s_v, d_v)
    _compute_keys(wid, s_v, d_v, k_v)
    _gather_rows(t_hbm, k_v, o_v, sem)
    z16 = jnp.zeros((LANES,), jnp.int32)
    def zc(i, _):
        zc_v[pl.ds(i * LANES, LANES)] = z16
        return 0
    lax.fori_loop(0, CSL // LANES, zc, 0)
    cbase = tile * CSL
    pltpu.sync_copy(zc_v, cnts_sp.at[pl.ds(cbase, CSL)])
    iota = _iota16()
    def flg(i, _):
        g = i // 8
        o = (i % 8) * LANES
        o16 = o_v[g, pl.ds(o, LANES)]
        won = o16 == (id_base + base + i * LANES + iota)
        f_v[g, pl.ds(o, LANES)] = jnp.where(won, 1, 0)
        sidx_v[g, pl.ds(o, LANES)] = z16 + (cbase + g)
        return 0
    lax.fori_loop(0, NSL, flg, 0)
    for l in range(1, 8):
        f_v[NG - 1, pl.ds(l * LANES, LANES)] = z16
        sidx_v[NG - 1, pl.ds(l * LANES, LANES)] = z16 + (cbase + NG - 1)
    def cfire(g, _):
        pltpu.async_copy(f_v.at[g], cnts_sp.at[sidx_v.at[g]], sem, add=True)
        return 0
    lax.fori_loop(0, NG, cfire, 0)
    def cdrain(g, _):
        pltpu.make_async_copy(t_hbm.at[pl.ds(0, GRP)], k_v.at[g], sem).wait()
        return 0
    lax.fori_loop(0, NG, cdrain, 0)
    pltpu.sync_copy(cnts_sp.at[pl.ds(cbase, GRP)], gcnt_v)
    iota = _iota16()
    for l in range(8):
        o = l * LANES
        gv = gcnt_v[pl.ds(o, LANES)]
        o_v[0, pl.ds(o, LANES)] = jnp.where(o + iota < NG, gv, 0)
        sidx_v[0, pl.ds(o, LANES)] = z16 + (cbase + TOTSLOT)
    pltpu.sync_copy(o_v.at[0], cnts_sp.at[sidx_v.at[0]], add=True)
    pltpu.sync_copy(cnts_sp.at[pl.ds(cbase, GRP)], gcnt_v)
    gslices = [gcnt_v[pl.ds(k * LANES, LANES)] for k in range(8)]
    pos = jnp.int32(0)
    for g in range(NG):
        gc = gslices[g // LANES][g % LANES]
        @pl.when(gc > 0)
        def _(g=g, pos=pos):
            for l in range(8):
                o = l * LANES
                cs_v[pl.ds(pos * GRP + o, LANES)] = (
                    s_v[pl.ds(g * GRP + o, LANES)]
                    if g * GRP + o + LANES <= EPT
                    else z16)
                cd_v[pl.ds(pos * GRP + o, LANES)] = (
                    d_v[pl.ds(g * GRP + o, LANES)]
                    if g * GRP + o + LANES <= EPT
                    else z16)
                cf_v[pl.ds(pos * GRP + o, LANES)] = f_v[g, pl.ds(o, LANES)]
        pos = pos + jnp.where(gc > 0, 1, 0)
    pbase = wid * PEPT
    pltpu.sync_copy(cs_v.at[pl.ds(0, PEPT)], src_out.at[pl.ds(pbase, PEPT)])
    pltpu.sync_copy(cd_v.at[pl.ds(0, PEPT)], dst_out.at[pl.ds(pbase, PEPT)])
    pltpu.sync_copy(cf_v.at[pl.ds(0, PEPT)], val_out.at[pl.ds(pbase, PEPT)])
    cnt_v[pl.ds(0, LANES)] = z16 + pos * GRP
    pltpu.sync_copy(cnt_v, cnt_out.at[wid])
    tslice = gcnt_v[pl.ds((TOTSLOT // LANES) * LANES, LANES)]
    cnt_v[pl.ds(0, LANES)] = z16 + tslice[TOTSLOT % LANES]
    pltpu.sync_copy(cnt_v, tcnt_out.at[wid])


def _build_compact(id_base):
    return pl.kernel(
        functools.partial(_compact_kernel, id_base),
        out_type=(
            jax.ShapeDtypeStruct((NT * PEPT,), jnp.int32),
            jax.ShapeDtypeStruct((NT * PEPT,), jnp.int32),
            jax.ShapeDtypeStruct((NT * PEPT,), jnp.int32),
            jax.ShapeDtypeStruct((NT, LANES), jnp.int32),
            jax.ShapeDtypeStruct((NT, LANES), jnp.int32),
        ),
        mesh=_mesh,
        scratch_types=[
            pltpu.VMEM((EPT,), jnp.int32),
            pltpu.VMEM((EPT,), jnp.int32),
            pltpu.VMEM((NG, GRP), jnp.int32),
            pltpu.VMEM((NG, GRP), jnp.int32),
            pltpu.VMEM((NG, GRP), jnp.int32),
            pltpu.VMEM((NG, GRP), jnp.int32),
            pltpu.VMEM((PEPT,), jnp.int32),
            pltpu.VMEM((PEPT,), jnp.int32),
            pltpu.VMEM((PEPT,), jnp.int32),
            pltpu.VMEM((CSL,), jnp.int32),
            pltpu.VMEM((GRP,), jnp.int32),
            pltpu.VMEM((LANES,), jnp.int32),
            pltpu.VMEM_SHARED((NS * CSL,), jnp.int32),
            pltpu.SemaphoreType.DMA,
        ],
    )


def _zero_zb(zb_v):
    z16 = jnp.zeros((LANES,), jnp.float32)
    def body(i, _):
        zb_v[i // 8, pl.ds((i % 8) * LANES, LANES)] = z16
        return 0
    lax.fori_loop(0, 8 * 8, body, 0)


def _zero_acc(acc, tile, zb_v):
    start = tile * (HROWS // NS)
    for j in range(HROWS // NS // 8):
        pltpu.sync_copy(zb_v, acc.at[pl.ds(start + j * 8, 8)])


def _get_cnt(cnt_hbm, r, cnt_v):
    pltpu.sync_copy(cnt_hbm.at[r], cnt_v)
    return cnt_v[pl.ds(0, LANES)][0]


def _prop_kernel(x_hbm, src_hbm, dst_hbm, val_hbm, cnt_hbm, out_hbm,
                 s_g, d_g, f_g, ss_v, sds_v, rows_v, zb_v, cnt_v, acc, sem):
    core = lax.axis_index("c")
    tile = lax.axis_index("s")
    _zero_zb(zb_v)
    r = core * NS + tile
    cnt = _get_cnt(cnt_hbm, r, cnt_v)
    pbase = r * PEPT
    iota = _iota16()
    _zero_acc(acc, tile, zb_v)
    plsc.subcore_barrier()
    def grp(g, _):
        @pl.when(g * GRP < cnt)
        def _():
            pltpu.sync_copy(src_hbm.at[pl.ds(pbase + g * GRP, GRP)], s_g)
            pltpu.sync_copy(dst_hbm.at[pl.ds(pbase + g * GRP, GRP)], d_g)
            pltpu.sync_copy(val_hbm.at[pl.ds(pbase + g * GRP, GRP)], f_g)
            for l in range(8):
                sl = pl.ds(l * LANES, LANES)
                fl = f_g[sl] > 0
                s16 = s_g[sl]
                d16 = d_g[sl]
                spread = (iota + l * LANES) & 63
                ss_v[sl] = jnp.where(fl, s16, spread)
                sds_v[sl] = jnp.where(fl, d16, N + spread)
            pltpu.async_copy(x_hbm.at[ss_v], rows_v, sem).wait()
            pltpu.sync_copy(rows_v, acc.at[sds_v], add=True)
        return 0
    lax.fori_loop(0, NG, grp, 0)
    plsc.subcore_barrier()
    rt = HROWS // NS
    pltpu.sync_copy(acc.at[pl.ds(tile * rt, rt)],
                    out_hbm.at[core, pl.ds(tile * rt, rt)])


def _build_prop():
    return pl.kernel(
        _prop_kernel,
        out_type=jax.ShapeDtypeStruct((NC, HROWS, D), jnp.float32),
        mesh=_mesh,
        scratch_types=[
            pltpu.VMEM((GRP,), jnp.int32),
            pltpu.VMEM((GRP,), jnp.int32),
            pltpu.VMEM((GRP,), jnp.int32),
            pltpu.VMEM((GRP,), jnp.int32),
            pltpu.VMEM((GRP,), jnp.int32),
            pltpu.VMEM((GRP, D), jnp.float32),
            pltpu.VMEM((8, D), jnp.float32),
            pltpu.VMEM((LANES,), jnp.int32),
            pltpu.VMEM_SHARED((HROWS, D), jnp.float32),
            pltpu.SemaphoreType.DMA,
        ],
    )


def _attn_kernel(x_hbm, xq_hbm, x2_hbm, src_hbm, dst_hbm, val_hbm, cnt_hbm,
                 out_hbm, s_g, d_g, f_g, ss_v, sdg_v, sds_v,
                 xqr_v, xdr_v, x2r_v, ra_v, att16_v, idxr_v, za_v, zb_v,
                 cnt_v, acc, attn_sp, sem):
    core = lax.axis_index("c")
    tile = lax.axis_index("s")
    _zero_zb(zb_v)
    z16f = jnp.zeros((LANES,), jnp.float32)
    def za(i, _):
        za_v[pl.ds(i * LANES, LANES)] = z16f
        return 0
    lax.fori_loop(0, ASL // LANES, za, 0)
    abase = tile * ASL
    z16i = jnp.zeros((LANES,), jnp.int32)
    def ip(i, _):
        b = i // 8
        o = i % 8
        idxr_v[b, pl.ds(o * LANES, LANES)] = z16i + (abase + 8 * b + o)
        return 0
    lax.fori_loop(0, 8 * 8, ip, 0)
    r = core * NS + tile
    cnt = _get_cnt(cnt_hbm, r, cnt_v)
    pbase = r * PEPT
    iota = _iota16()
    if True:
        _zero_acc(acc, tile, zb_v)
        plsc.subcore_barrier()
        def grp(g, _):
            @pl.when(g * G2 < cnt)
            def _():
                pltpu.sync_copy(src_hbm.at[pl.ds(pbase + g * G2, G2)], s_g)
                pltpu.sync_copy(dst_hbm.at[pl.ds(pbase + g * G2, G2)], d_g)
                pltpu.sync_copy(val_hbm.at[pl.ds(pbase + g * G2, G2)], f_g)
                for l in range(4):
                    sl = pl.ds(l * LANES, LANES)
                    fl = f_g[sl] > 0
                    s16 = s_g[sl]
                    d16 = d_g[sl]
                    spread = (iota + l * LANES) & 63
                    ss_v[sl] = jnp.where(fl, s16, spread)
                    sdg_v[sl] = jnp.where(fl, d16, spread)
                    sds_v[sl] = jnp.where(fl, d16, N + spread)
                cp1 = pltpu.async_copy(xq_hbm.at[ss_v], xqr_v, sem)
                cp2 = pltpu.async_copy(x_hbm.at[sdg_v], xdr_v, sem)
                cp3 = pltpu.async_copy(x2_hbm.at[ss_v], x2r_v, sem)
                cp1.wait()
                cp2.wait()
                cp3.wait()
                def prod(rr, _):
                    a16 = (xqr_v[rr, pl.ds(0, LANES)]
                           * xdr_v[rr, pl.ds(0, LANES)])
                    for l in range(1, 8):
                        a16 = a16 + (xqr_v[rr, pl.ds(l * LANES, LANES)]
                                     * xdr_v[rr, pl.ds(l * LANES, LANES)])
                    ra_v[pl.ds(rr * LANES, LANES)] = a16
                    return 0
                lax.fori_loop(0, G2, prod, 0)
                pltpu.sync_copy(za_v, attn_sp.at[pl.ds(abase, ASL)])
                def afire(b, _):
                    pltpu.async_copy(ra_v.at[pl.ds(b * GRP, GRP)],
                                     attn_sp.at[idxr_v.at[b]], sem, add=True)
                    return 0
                lax.fori_loop(0, 8, afire, 0)
                def adrain(b, _):
                    pltpu.make_async_copy(x_hbm.at[0], xdr_v.at[0], sem).wait()
                    return 0
                lax.fori_loop(0, 8, adrain, 0)
                def bfire(b, _):
                    pltpu.async_copy(attn_sp.at[idxr_v.at[b]],
                                     att16_v.at[pl.ds(b * GRP, GRP)], sem)
                    return 0
                lax.fori_loop(0, 8, bfire, 0)
                def bdrain(b, _):
                    pltpu.make_async_copy(x_hbm.at[0], xdr_v.at[0], sem).wait()
                    return 0
                lax.fori_loop(0, 8, bdrain, 0)
                def ymul(rr, _):
                    at16 = att16_v[pl.ds(rr * LANES, LANES)]
                    for l in range(8):
                        xdr_v[rr, pl.ds(l * LANES, LANES)] = (
                            at16 * x2r_v[rr, pl.ds(l * LANES, LANES)])
                    return 0
                lax.fori_loop(0, G2, ymul, 0)
                pltpu.sync_copy(xdr_v, acc.at[sds_v], add=True)
            return 0
        lax.fori_loop(0, NG2, grp, 0)
        plsc.subcore_barrier()
        rt = HROWS // NS
        pltpu.sync_copy(acc.at[pl.ds(tile * rt, rt)],
                        out_hbm.at[core, pl.ds(tile * rt, rt)])


def _build_attn():
    return pl.kernel(
        _attn_kernel,
        out_type=jax.ShapeDtypeStruct((NC, HROWS, D), jnp.float32),
        mesh=_mesh,
        scratch_types=[
            pltpu.VMEM((G2,), jnp.int32),
            pltpu.VMEM((G2,), jnp.int32),
            pltpu.VMEM((G2,), jnp.int32),
            pltpu.VMEM((G2,), jnp.int32),
            pltpu.VMEM((G2,), jnp.int32),
            pltpu.VMEM((G2,), jnp.int32),
            pltpu.VMEM((G2, D), jnp.float32),
            pltpu.VMEM((G2, D), jnp.float32),
            pltpu.VMEM((G2, D), jnp.float32),
            pltpu.VMEM((G2 * LANES,), jnp.float32),
            pltpu.VMEM((G2 * LANES,), jnp.float32),
            pltpu.VMEM((8, GRP), jnp.int32),
            pltpu.VMEM((ASL,), jnp.float32),
            pltpu.VMEM((8, D), jnp.float32),
            pltpu.VMEM((LANES,), jnp.int32),
            pltpu.VMEM_SHARED((HROWS, D), jnp.float32),
            pltpu.VMEM_SHARED((NS * ASL,), jnp.float32),
            pltpu.SemaphoreType.DMA,
        ],
    )


_RB = 1000


def _t1_body(x_ref, p0_ref, p1_ref, w1_ref, q_ref, w2_ref,
             x1_ref, xq_ref, x2_ref):
    x = x_ref[...]
    p = p0_ref[...] + p1_ref[...]
    w1 = w1_ref[...]
    q = q_ref[...]
    h = jnp.dot(p, w1.T, preferred_element_type=jnp.float32)
    x1_ref[...] = x + jnp.dot(h, q, preferred_element_type=jnp.float32) * (
        float(N) / float(E))
    xq_ref[...] = jnp.dot(x, q, preferred_element_type=jnp.float32)
    x2_ref[...] = jnp.dot(x, w2_ref[...].T, preferred_element_type=jnp.float32)


def _t1(x, p0, p1, w1, q, w2):
    full = pl.BlockSpec((D, D), lambda i: (0, 0))
    blk = pl.BlockSpec((_RB, D), lambda i: (i, 0))
    return pl.pallas_call(
        _t1_body,
        grid=(N // _RB,),
        in_specs=[blk, blk, blk, full, full, full],
        out_specs=[blk, blk, blk],
        out_shape=[jax.ShapeDtypeStruct((N, D), jnp.float32)] * 3,
    )(x, p0, p1, w1, q, w2)


def _t2_body(x1_ref, a0_ref, a1_ref, q_ref, cnt_ref, out_ref):
    a = a0_ref[...] + a1_ref[...]
    cnt2 = jnp.sum(cnt_ref[...][:, 0]).astype(jnp.float32)
    scale = float(N) / cnt2
    out_ref[...] = x1_ref[...] - jnp.dot(
        a, q_ref[...], preferred_element_type=jnp.float32) * scale


def _t2(x1, a0, a1, q, cnt2):
    full = pl.BlockSpec((D, D), lambda i: (0, 0))
    blk = pl.BlockSpec((_RB, D), lambda i: (i, 0))
    cblk = pl.BlockSpec((NT, LANES), lambda i: (0, 0))
    return pl.pallas_call(
        _t2_body,
        grid=(N // _RB,),
        in_specs=[blk, blk, blk, full, cblk],
        out_specs=blk,
        out_shape=jax.ShapeDtypeStruct((N, D), jnp.float32),
    )(x1, a0, a1, q, cnt2)


def kernel(input, edge_index, edge_index_2, Q, mask, W1_0, W2_0, W1_1, W2_1):
    ta = jnp.full((TSZ,), -1, jnp.int32)
    tb = jnp.full((TSZ,), -1, jnp.int32)
    mask_f = mask.reshape(-1)
    e1_f = edge_index.reshape(-1)
    e2_f = edge_index_2.reshape(-1)
    e1_k = _build_e1()
    toka = e1_k(ta, mask_f)
    tokb = e1_k(tb, mask_f)
    claim_k = _build_claim(0)
    compact_k = _build_compact(0)
    tok2 = claim_k(ta, e1_f, toka)
    src1, dst1, val1, cnt1, _tc1 = compact_k(ta, e1_f, tok2)
    tok4 = claim_k(tb, e2_f, tokb)
    src2, dst2, val2, cnt2, tcnt2 = compact_k(tb, e2_f, tok4)

    prop_k = _build_prop()
    attn_k = _build_attn()

    x = input
    for (w1, w2) in ((W1_0, W2_0), (W1_1, W2_1)):
        prop = prop_k(x, src1, dst1, val1, cnt1)
        x1, xq, x2 = _t1(x, prop[0, :N], prop[1, :N], w1, Q, w2)
        agg = attn_k(x, xq, x2, src2, dst2, val2, cnt2)
        x = _t2(x1, agg[0, :N], agg[1, :N], Q, tcnt2)
    return x


# speedup vs baseline: 39.3143x; 2.3094x over previous
import functools

import jax
import jax.numpy as jnp
from jax import lax
from jax.experimental import pallas as pl
from jax.experimental.pallas import tpu as pltpu
from jax.experimental.pallas import tpu_sc as plsc

NC = 2
NS = 16
NT = NC * NS
LANES = 16

N = 10000
D = 128
E = 320000
EPT = E // NT
NSL = EPT // LANES
GRP = 128
NG = (EPT + GRP - 1) // GRP
DUMP = N * N
TSZ = N * N + NT * (NG * GRP) + 64
PEPT = NG * GRP
CSL = 656
TOTSLOT = 100
HROWS = 10240
G2 = 64
NG2 = PEPT // G2
ASL = 160

_mesh = plsc.VectorSubcoreMesh(core_axis_name="c", subcore_axis_name="s")


def _wid():
    return lax.axis_index("c") * NS + lax.axis_index("s")


def _iota16():
    return lax.broadcasted_iota(jnp.int32, (LANES,), 0)


def _dump16(wid, l):
    return DUMP + wid * PEPT + l * LANES + _iota16()


def _compute_keys(wid, s_v, d_v, k_v):
    def body(i, _):
        g = i // 8
        o = (i % 8) * LANES
        s16 = s_v[pl.ds(i * LANES, LANES)]
        d16 = d_v[pl.ds(i * LANES, LANES)]
        k_v[g, pl.ds(o, LANES)] = s16 * N + d16
        return 0
    lax.fori_loop(0, NSL, body, 0)
    for l in range(1, 8):
        k_v[NG - 1, pl.ds(l * LANES, LANES)] = _dump16(wid, l)


def _load_pair(pair_hbm, base, s_v, d_v):
    pltpu.sync_copy(pair_hbm.at[pl.ds(base, EPT)], s_v)
    pltpu.sync_copy(pair_hbm.at[pl.ds(E + base, EPT)], d_v)


def _gather_rows(t_hbm, k_v, o_v, sem):
    def fire(g, _):
        pltpu.async_copy(t_hbm.at[k_v.at[g]], o_v.at[g], sem)
        return 0
    lax.fori_loop(0, NG, fire, 0)
    def drain(g, _):
        pltpu.make_async_copy(t_hbm.at[pl.ds(0, GRP)], o_v.at[g], sem).wait()
        return 0
    lax.fori_loop(0, NG, drain, 0)


def _mask_kernel(t_hbm, mask_hbm, tok_out, s_v, d_v, k_v, val_v, sem):
    wid = _wid()
    base = wid * EPT
    _load_pair(mask_hbm, base, s_v, d_v)
    _compute_keys(wid, s_v, d_v, k_v)
    m2 = jnp.full((LANES,), -2, jnp.int32)
    for l in range(8):
        val_v[pl.ds(l * LANES, LANES)] = m2
    def fire(g, _):
        pltpu.async_copy(val_v, t_hbm.at[k_v.at[g]], sem)
        return 0
    lax.fori_loop(0, NG, fire, 0)
    def drain(g, _):
        pltpu.make_async_copy(t_hbm.at[pl.ds(0, GRP)], k_v.at[g], sem).wait()
        return 0
    lax.fori_loop(0, NG, drain, 0)
    @pl.when(_wid() == 0)
    def _():
        val_v[pl.ds(0, LANES)] = jnp.zeros((LANES,), jnp.int32)
        pltpu.sync_copy(val_v.at[pl.ds(0, LANES)], tok_out)


def _build_e1():
    return pl.kernel(
        _mask_kernel,
        out_type=jax.ShapeDtypeStruct((LANES,), jnp.int32),
        mesh=_mesh,
        scratch_types=[
            pltpu.VMEM((EPT,), jnp.int32),
            pltpu.VMEM((EPT,), jnp.int32),
            pltpu.VMEM((NG, GRP), jnp.int32),
            pltpu.VMEM((GRP,), jnp.int32),
            pltpu.SemaphoreType.DMA,
        ],
    )


def _claim_kernel(id_base, t_hbm, ei_hbm, tok_in, tok_out,
                  s_v, d_v, k_v, m_v, q_v, id_v, sem):
    del tok_in
    wid = _wid()
    base = wid * EPT
    _load_pair(ei_hbm, base, s_v, d_v)
    _compute_keys(wid, s_v, d_v, k_v)
    _gather_rows(t_hbm, k_v, m_v, sem)
    iota = _iota16()
    def sel(i, _):
        g = i // 8
        o = (i % 8) * LANES
        m16 = m_v[g, pl.ds(o, LANES)]
        k16 = k_v[g, pl.ds(o, LANES)]
        q_v[g, pl.ds(o, LANES)] = jnp.where(
            m16 != -1, k16, DUMP + wid * PEPT + i * LANES + iota)
        p = i * LANES + iota
        id_v[g, pl.ds(o, LANES)] = id_base + base + p
        return 0
    lax.fori_loop(0, NSL, sel, 0)
    m4 = jnp.full((LANES,), -4, jnp.int32)
    for l in range(1, 8):
        q_v[NG - 1, pl.ds(l * LANES, LANES)] = _dump16(wid, l)
        id_v[NG - 1, pl.ds(l * LANES, LANES)] = m4
    def fire2(g, _):
        pltpu.async_copy(id_v.at[g], t_hbm.at[q_v.at[g]], sem)
        return 0
    lax.fori_loop(0, NG, fire2, 0)
    def drain2(g, _):
        pltpu.make_async_copy(t_hbm.at[pl.ds(0, GRP)], m_v.at[g], sem).wait()
        return 0
    lax.fori_loop(0, NG, drain2, 0)
    @pl.when(wid == 0)
    def _():
        id_v[0, pl.ds(0, LANES)] = jnp.zeros((LANES,), jnp.int32)
        pltpu.sync_copy(id_v.at[0, pl.ds(0, LANES)], tok_out)


def _build_claim(id_base):
    return pl.kernel(
        functools.partial(_claim_kernel, id_base),
        out_type=jax.ShapeDtypeStruct((LANES,), jnp.int32),
        mesh=_mesh,
        scratch_types=[
            pltpu.VMEM((EPT,), jnp.int32),
            pltpu.VMEM((EPT,), jnp.int32),
            pltpu.VMEM((NG, GRP), jnp.int32),
            pltpu.VMEM((NG, GRP), jnp.int32),
            pltpu.VMEM((NG, GRP), jnp.int32),
            pltpu.VMEM((NG, GRP), jnp.int32),
            pltpu.SemaphoreType.DMA,
        ],
    )


def _compact_kernel(id_base, t_hbm, ei_hbm, tok_in, src_out, dst_out, val_out,
                    cnt_out, tcnt_out, s_v, d_v, k_v, o_v, f_v, sidx_v,
                    cs_v, cd_v, cf_v, zc_v, gcnt_v, cnt_v, cnts_sp, sem):
    del tok_in
    wid = _wid()
    tile = lax.axis_index("s")
    base = wid * EPT
    _load_pair(ei_hbm, base, s_v, d_v)
    _compute_keys(wid, s_v, d_v, k_v)
    _gather_rows(t_hbm, k_v, o_v, sem)
    z16 = jnp.zeros((LANES,), jnp.int32)
    def zc(i, _):
        zc_v[pl.ds(i * LANES, LANES)] = z16
        return 0
    lax.fori_loop(0, CSL // LANES, zc, 0)
    cbase = tile * CSL
    pltpu.sync_copy(zc_v, cnts_sp.at[pl.ds(cbase, CSL)])
    iota = _iota16()
    def flg(i, _):
        g = i // 8
        o = (i % 8) * LANES
        o16 = o_v[g, pl.ds(o, LANES)]
        won = o16 == (id_base + base + i * LANES + iota)
        f_v[g, pl.ds(o, LANES)] = jnp.where(won, 1, 0)
        sidx_v[g, pl.ds(o, LANES)] = z16 + (cbase + g)
        return 0
    lax.fori_loop(0, NSL, flg, 0)
    for l in range(1, 8):
        f_v[NG - 1, pl.ds(l * LANES, LANES)] = z16
        sidx_v[NG - 1, pl.ds(l * LANES, LANES)] = z16 + (cbase + NG - 1)
    def cfire(g, _):
        pltpu.async_copy(f_v.at[g], cnts_sp.at[sidx_v.at[g]], sem, add=True)
        return 0
    lax.fori_loop(0, NG, cfire, 0)
    def cdrain(g, _):
        pltpu.make_async_copy(t_hbm.at[pl.ds(0, GRP)], k_v.at[g], sem).wait()
        return 0
    lax.fori_loop(0, NG, cdrain, 0)
    pltpu.sync_copy(cnts_sp.at[pl.ds(cbase, GRP)], gcnt_v)
    iota = _iota16()
    for l in range(8):
        o = l * LANES
        gv = gcnt_v[pl.ds(o, LANES)]
        o_v[0, pl.ds(o, LANES)] = jnp.where(o + iota < NG, gv, 0)
        sidx_v[0, pl.ds(o, LANES)] = z16 + (cbase + TOTSLOT)
    pltpu.sync_copy(o_v.at[0], cnts_sp.at[sidx_v.at[0]], add=True)
    pltpu.sync_copy(cnts_sp.at[pl.ds(cbase, GRP)], gcnt_v)
    gslices = [gcnt_v[pl.ds(k * LANES, LANES)] for k in range(8)]
    pos = jnp.int32(0)
    for g in range(NG):
        gc = gslices[g // LANES][g % LANES]
        @pl.when(gc > 0)
        def _(g=g, pos=pos):
            for l in range(8):
                o = l * LANES
                cs_v[pl.ds(pos * GRP + o, LANES)] = (
                    s_v[pl.ds(g * GRP + o, LANES)]
                    if g * GRP + o + LANES <= EPT
                    else z16)
                cd_v[pl.ds(pos * GRP + o, LANES)] = (
                    d_v[pl.ds(g * GRP + o, LANES)]
                    if g * GRP + o + LANES <= EPT
                    else z16)
                cf_v[pl.ds(pos * GRP + o, LANES)] = f_v[g, pl.ds(o, LANES)]
        pos = pos + jnp.where(gc > 0, 1, 0)
    pbase = wid * PEPT
    pltpu.sync_copy(cs_v.at[pl.ds(0, PEPT)], src_out.at[pl.ds(pbase, PEPT)])
    pltpu.sync_copy(cd_v.at[pl.ds(0, PEPT)], dst_out.at[pl.ds(pbase, PEPT)])
    pltpu.sync_copy(cf_v.at[pl.ds(0, PEPT)], val_out.at[pl.ds(pbase, PEPT)])
    cnt_v[pl.ds(0, LANES)] = z16 + pos * GRP
    pltpu.sync_copy(cnt_v, cnt_out.at[wid])
    tslice = gcnt_v[pl.ds((TOTSLOT // LANES) * LANES, LANES)]
    cnt_v[pl.ds(0, LANES)] = z16 + tslice[TOTSLOT % LANES]
    pltpu.sync_copy(cnt_v, tcnt_out.at[wid])


def _build_compact(id_base):
    return pl.kernel(
        functools.partial(_compact_kernel, id_base),
        out_type=(
            jax.ShapeDtypeStruct((NT * PEPT,), jnp.int32),
            jax.ShapeDtypeStruct((NT * PEPT,), jnp.int32),
            jax.ShapeDtypeStruct((NT * PEPT,), jnp.int32),
            jax.ShapeDtypeStruct((NT, LANES), jnp.int32),
            jax.ShapeDtypeStruct((NT, LANES), jnp.int32),
        ),
        mesh=_mesh,
        scratch_types=[
            pltpu.VMEM((EPT,), jnp.int32),
            pltpu.VMEM((EPT,), jnp.int32),
            pltpu.VMEM((NG, GRP), jnp.int32),
            pltpu.VMEM((NG, GRP), jnp.int32),
            pltpu.VMEM((NG, GRP), jnp.int32),
            pltpu.VMEM((NG, GRP), jnp.int32),
            pltpu.VMEM((PEPT,), jnp.int32),
            pltpu.VMEM((PEPT,), jnp.int32),
            pltpu.VMEM((PEPT,), jnp.int32),
            pltpu.VMEM((CSL,), jnp.int32),
            pltpu.VMEM((GRP,), jnp.int32),
            pltpu.VMEM((LANES,), jnp.int32),
            pltpu.VMEM_SHARED((NS * CSL,), jnp.int32),
            pltpu.SemaphoreType.DMA,
        ],
    )


def _zero_zb(zb_v):
    z16 = jnp.zeros((LANES,), jnp.float32)
    def body(i, _):
        zb_v[i // 8, pl.ds((i % 8) * LANES, LANES)] = z16
        return 0
    lax.fori_loop(0, 8 * 8, body, 0)


def _zero_acc(acc, tile, zb_v):
    start = tile * (HROWS // NS)
    for j in range(HROWS // NS // 8):
        pltpu.sync_copy(zb_v, acc.at[pl.ds(start + j * 8, 8)])


def _get_cnt(cnt_hbm, r, cnt_v):
    pltpu.sync_copy(cnt_hbm.at[r], cnt_v)
    return cnt_v[pl.ds(0, LANES)][0]


def _prop_kernel(x_hbm, src_hbm, dst_hbm, val_hbm, cnt_hbm, out_hbm,
                 s_g, d_g, f_g, ss_v, sds_v, rows_v, zb_v, cnt_v, acc, sem):
    core = lax.axis_index("c")
    tile = lax.axis_index("s")
    _zero_zb(zb_v)
    r = core * NS + tile
    cnt = _get_cnt(cnt_hbm, r, cnt_v)
    pbase = r * PEPT
    iota = _iota16()
    _zero_acc(acc, tile, zb_v)
    plsc.subcore_barrier()
    def grp(g, _):
        @pl.when(g * GRP < cnt)
        def _():
            pltpu.sync_copy(src_hbm.at[pl.ds(pbase + g * GRP, GRP)], s_g)
            pltpu.sync_copy(dst_hbm.at[pl.ds(pbase + g * GRP, GRP)], d_g)
            pltpu.sync_copy(val_hbm.at[pl.ds(pbase + g * GRP, GRP)], f_g)
            for l in range(8):
                sl = pl.ds(l * LANES, LANES)
                fl = f_g[sl] > 0
                s16 = s_g[sl]
                d16 = d_g[sl]
                spread = (iota + l * LANES) & 63
                ss_v[sl] = jnp.where(fl, s16, spread)
                sds_v[sl] = jnp.where(fl, d16, N + spread)
            pltpu.async_copy(x_hbm.at[ss_v], rows_v, sem).wait()
            pltpu.sync_copy(rows_v, acc.at[sds_v], add=True)
        return 0
    lax.fori_loop(0, NG, grp, 0)
    plsc.subcore_barrier()
    rt = HROWS // NS
    pltpu.sync_copy(acc.at[pl.ds(tile * rt, rt)],
                    out_hbm.at[core, pl.ds(tile * rt, rt)])


def _build_prop():
    return pl.kernel(
        _prop_kernel,
        out_type=jax.ShapeDtypeStruct((NC, HROWS, D), jnp.float32),
        mesh=_mesh,
        scratch_types=[
            pltpu.VMEM((GRP,), jnp.int32),
            pltpu.VMEM((GRP,), jnp.int32),
            pltpu.VMEM((GRP,), jnp.int32),
            pltpu.VMEM((GRP,), jnp.int32),
            pltpu.VMEM((GRP,), jnp.int32),
            pltpu.VMEM((GRP, D), jnp.float32),
            pltpu.VMEM((8, D), jnp.float32),
            pltpu.VMEM((LANES,), jnp.int32),
            pltpu.VMEM_SHARED((HROWS, D), jnp.float32),
            pltpu.SemaphoreType.DMA,
        ],
    )


def _attn_kernel(x_hbm, xq_hbm, x2_hbm, src_hbm, dst_hbm, val_hbm, cnt_hbm,
                 out_hbm, s_g, d_g, f_g, ss_v, sdg_v, sds_v,
                 xqr_v, xdr_v, x2r_v, ra_v, att16_v, idxr_v, za_v, zb_v,
                 cnt_v, acc, attn_sp, sem):
    core = lax.axis_index("c")
    tile = lax.axis_index("s")
    _zero_zb(zb_v)
    z16f = jnp.zeros((LANES,), jnp.float32)
    def za(i, _):
        za_v[pl.ds(i * LANES, LANES)] = z16f
        return 0
    lax.fori_loop(0, ASL // LANES, za, 0)
    abase = tile * ASL
    z16i = jnp.zeros((LANES,), jnp.int32)
    def ip(i, _):
        b = i // 8
        o = i % 8
        idxr_v[b, pl.ds(o * LANES, LANES)] = z16i + (abase + 8 * b + o)
        return 0
    lax.fori_loop(0, 8 * 8, ip, 0)
    r = core * NS + tile
    cnt = _get_cnt(cnt_hbm, r, cnt_v)
    pbase = r * PEPT
    iota = _iota16()
    if True:
        _zero_acc(acc, tile, zb_v)
        plsc.subcore_barrier()
        def grp(g, _):
            @pl.when(g * G2 < cnt)
            def _():
                pltpu.sync_copy(src_hbm.at[pl.ds(pbase + g * G2, G2)], s_g)
                pltpu.sync_copy(dst_hbm.at[pl.ds(pbase + g * G2, G2)], d_g)
                pltpu.sync_copy(val_hbm.at[pl.ds(pbase + g * G2, G2)], f_g)
                for l in range(4):
                    sl = pl.ds(l * LANES, LANES)
                    fl = f_g[sl] > 0
                    s16 = s_g[sl]
                    d16 = d_g[sl]
                    spread = (iota + l * LANES) & 63
                    ss_v[sl] = jnp.where(fl, s16, spread)
                    sdg_v[sl] = jnp.where(fl, d16, spread)
                    sds_v[sl] = jnp.where(fl, d16, N + spread)
                cp1 = pltpu.async_copy(xq_hbm.at[ss_v], xqr_v, sem)
                cp2 = pltpu.async_copy(x_hbm.at[sdg_v], xdr_v, sem)
                cp3 = pltpu.async_copy(x2_hbm.at[ss_v], x2r_v, sem)
                cp1.wait()
                cp2.wait()
                cp3.wait()
                def prod(rr, _):
                    a16 = (xqr_v[rr, pl.ds(0, LANES)]
                           * xdr_v[rr, pl.ds(0, LANES)])
                    for l in range(1, 8):
                        a16 = a16 + (xqr_v[rr, pl.ds(l * LANES, LANES)]
                                     * xdr_v[rr, pl.ds(l * LANES, LANES)])
                    ra_v[pl.ds(rr * LANES, LANES)] = a16
                    return 0
                lax.fori_loop(0, G2, prod, 0)
                pltpu.sync_copy(za_v, attn_sp.at[pl.ds(abase, ASL)])
                def afire(b, _):
                    pltpu.async_copy(ra_v.at[pl.ds(b * GRP, GRP)],
                                     attn_sp.at[idxr_v.at[b]], sem, add=True)
                    return 0
                lax.fori_loop(0, 8, afire, 0)
                def adrain(b, _):
                    pltpu.make_async_copy(x_hbm.at[0], xdr_v.at[0], sem).wait()
                    return 0
                lax.fori_loop(0, 8, adrain, 0)
                def bfire(b, _):
                    pltpu.async_copy(attn_sp.at[idxr_v.at[b]],
                                     att16_v.at[pl.ds(b * GRP, GRP)], sem)
                    return 0
                lax.fori_loop(0, 8, bfire, 0)
                def bdrain(b, _):
                    pltpu.make_async_copy(x_hbm.at[0], xdr_v.at[0], sem).wait()
                    return 0
                lax.fori_loop(0, 8, bdrain, 0)
                def ymul(rr, _):
                    at16 = att16_v[pl.ds(rr * LANES, LANES)]
                    for l in range(8):
                        xdr_v[rr, pl.ds(l * LANES, LANES)] = (
                            at16 * x2r_v[rr, pl.ds(l * LANES, LANES)])
                    return 0
                lax.fori_loop(0, G2, ymul, 0)
                pltpu.sync_copy(xdr_v, acc.at[sds_v], add=True)
            return 0
        lax.fori_loop(0, NG2, grp, 0)
        plsc.subcore_barrier()
        rt = HROWS // NS
        pltpu.sync_copy(acc.at[pl.ds(tile * rt, rt)],
                        out_hbm.at[core, pl.ds(tile * rt, rt)])


def _build_attn():
    return pl.kernel(
        _attn_kernel,
        out_type=jax.ShapeDtypeStruct((NC, HROWS, D), jnp.float32),
        mesh=_mesh,
        scratch_types=[
            pltpu.VMEM((G2,), jnp.int32),
            pltpu.VMEM((G2,), jnp.int32),
            pltpu.VMEM((G2,), jnp.int32),
            pltpu.VMEM((G2,), jnp.int32),
            pltpu.VMEM((G2,), jnp.int32),
            pltpu.VMEM((G2,), jnp.int32),
            pltpu.VMEM((G2, D), jnp.float32),
            pltpu.VMEM((G2, D), jnp.float32),
            pltpu.VMEM((G2, D), jnp.float32),
            pltpu.VMEM((G2 * LANES,), jnp.float32),
            pltpu.VMEM((G2 * LANES,), jnp.float32),
            pltpu.VMEM((8, GRP), jnp.int32),
            pltpu.VMEM((ASL,), jnp.float32),
            pltpu.VMEM((8, D), jnp.float32),
            pltpu.VMEM((LANES,), jnp.int32),
            pltpu.VMEM_SHARED((HROWS, D), jnp.float32),
            pltpu.VMEM_SHARED((NS * ASL,), jnp.float32),
            pltpu.SemaphoreType.DMA,
        ],
    )


_RB = 1000


def _t1_body(x_ref, p0_ref, p1_ref, w1_ref, q_ref, w2_ref,
             x1_ref, xq_ref, x2_ref):
    x = x_ref[...]
    p = p0_ref[...] + p1_ref[...]
    w1 = w1_ref[...]
    q = q_ref[...]
    h = jnp.dot(p, w1.T, preferred_element_type=jnp.float32)
    x1_ref[...] = x + jnp.dot(h, q, preferred_element_type=jnp.float32) * (
        float(N) / float(E))
    xq_ref[...] = jnp.dot(x, q, preferred_element_type=jnp.float32)
    x2_ref[...] = jnp.dot(x, w2_ref[...].T, preferred_element_type=jnp.float32)


def _t1(x, p0, p1, w1, q, w2):
    full = pl.BlockSpec((D, D), lambda i: (0, 0))
    blk = pl.BlockSpec((_RB, D), lambda i: (i, 0))
    return pl.pallas_call(
        _t1_body,
        grid=(N // _RB,),
        in_specs=[blk, blk, blk, full, full, full],
        out_specs=[blk, blk, blk],
        out_shape=[jax.ShapeDtypeStruct((N, D), jnp.float32)] * 3,
    )(x, p0, p1, w1, q, w2)


def _t2_body(x1_ref, a0_ref, a1_ref, q_ref, cnt_ref, out_ref):
    a = a0_ref[...] + a1_ref[...]
    cnt2 = jnp.sum(cnt_ref[...][:, 0]).astype(jnp.float32)
    scale = float(N) / cnt2
    out_ref[...] = x1_ref[...] - jnp.dot(
        a, q_ref[...], preferred_element_type=jnp.float32) * scale


def _t2(x1, a0, a1, q, cnt2):
    full = pl.BlockSpec((D, D), lambda i: (0, 0))
    blk = pl.BlockSpec((_RB, D), lambda i: (i, 0))
    cblk = pl.BlockSpec((NT, LANES), lambda i: (0, 0))
    return pl.pallas_call(
        _t2_body,
        grid=(N // _RB,),
        in_specs=[blk, blk, blk, full, cblk],
        out_specs=blk,
        out_shape=jax.ShapeDtypeStruct((N, D), jnp.float32),
    )(x1, a0, a1, q, cnt2)


def kernel(input, edge_index, edge_index_2, Q, mask, W1_0, W2_0, W1_1, W2_1):
    ta = jnp.full((TSZ,), -1, jnp.int32)
    tb = jnp.full((TSZ,), -1, jnp.int32)
    mask_f = mask.reshape(-1)
    e1_f = edge_index.reshape(-1)
    e2_f = edge_index_2.reshape(-1)
    e1_k = _build_e1()
    toka = e1_k(ta, mask_f)
    tokb = e1_k(tb, mask_f)
    claim_k = _build_claim(0)
    compact_k = _build_compact(0)
    tok2 = claim_k(ta, e1_f, toka)
    src1, dst1, val1, cnt1, _tc1 = compact_k(ta, e1_f, tok2)
    tok4 = claim_k(tb, e2_f, tokb)
    src2, dst2, val2, cnt2, tcnt2 = compact_k(tb, e2_f, tok4)

    prop_k = _build_prop()
    attn_k = _build_attn()

    x = input
    for (w1, w2) in ((W1_0, W2_0), (W1_1, W2_1)):
        prop = prop_k(x, src1, dst1, val1, cnt1)
        x1, xq, x2 = _t1(x, prop[0, :N], prop[1, :N], w1, Q, w2)
        agg = attn_k(x, xq, x2, src2, dst2, val2, cnt2)
        x = _t2(x1, agg[0, :N], agg[1, :N], Q, tcnt2)
    return x
